# Initial kernel scaffold; baseline (speedup 1.0000x reference)
import jax
import jax.numpy as jnp
from jax.experimental import pallas as pl


def kernel(x, pos, edge_index, batch, edge_attr, msg_W0, msg_b0, msg_W1, msg_b1, msg_W2, msg_b2, coord_W, coord_b, node_W0, node_b0, node_W1, node_b1, node_W2, node_b2, out_W, out_b):
    raise NotImplementedError("write your pallas kernel here")


# speedup vs baseline: 1.2345x; 1.2345x over previous
import functools

import jax
import jax.numpy as jnp
from jax import lax
from jax.experimental import pallas as pl
from jax.experimental.pallas import tpu as pltpu


def _lrelu(v):
    return jnp.maximum(v, 0.2 * v)


def _proj_body(x_ref, wd_ref, ws_ref, a_ref, b_ref):
    x = x_ref[...]
    a_ref[...] = jnp.dot(x, wd_ref[...], preferred_element_type=jnp.float32)
    b_ref[...] = jnp.dot(x, ws_ref[...], preferred_element_type=jnp.float32)


def _node_proj(x, wd, ws, nb):
    n, d = x.shape
    h = wd.shape[1]
    grid = n // nb
    return pl.pallas_call(
        _proj_body,
        grid=(grid,),
        in_specs=[
            pl.BlockSpec((nb, d), lambda i: (i, 0)),
            pl.BlockSpec((d, h), lambda i: (0, 0)),
            pl.BlockSpec((d, h), lambda i: (0, 0)),
        ],
        out_specs=[
            pl.BlockSpec((nb, h), lambda i: (i, 0)),
            pl.BlockSpec((nb, h), lambda i: (i, 0)),
        ],
        out_shape=[
            jax.ShapeDtypeStruct((n, h), jnp.float32),
            jax.ShapeDtypeStruct((n, h), jnp.float32),
        ],
    )(x, wd, ws)


def _edge_body(g1_ref, d4_ref, ea_ref, wsm_ref, wea_ref, b0_ref, w1_ref,
               b1_ref, w2_ref, b2_ref, cw_ref, cb_ref, offs_ref, s_ref,
               out_ref):
    d4 = d4_ref[...]
    dd = jnp.sum(d4 * d4, axis=1, keepdims=True)
    dist = jnp.sqrt(dd)
    s = s_ref[0, 0]
    t = dist * s - offs_ref[...]
    sm = jnp.exp(-(t * t))
    pre0 = (g1_ref[...]
            + jnp.dot(sm, wsm_ref[...], preferred_element_type=jnp.float32)
            + jnp.dot(ea_ref[...], wea_ref[...], preferred_element_type=jnp.float32)
            + b0_ref[...])
    h1 = _lrelu(pre0)
    h2 = _lrelu(jnp.dot(h1, w1_ref[...], preferred_element_type=jnp.float32)
                + b1_ref[...])
    z = jnp.dot(h2, w2_ref[...], preferred_element_type=jnp.float32) + b2_ref[...]
    msg = _lrelu(z)
    ew = jnp.sum(msg * cw_ref[...], axis=1, keepdims=True) + cb_ref[0, 0]
    inv = ew / (dist + 1e-6)
    wv = d4[:, :3] * inv
    eb = msg.shape[0]
    out_ref[...] = jnp.concatenate(
        [msg, wv, jnp.ones((eb, 1), jnp.float32),
         jnp.zeros((eb, 12), jnp.float32)], axis=1)


def _edge_mlp(g1, d4, ea, wsm, wea, b0, w1, b1, w2, b2, cw, cb, offs, s, eb):
    e = g1.shape[0]
    hh = g1.shape[1]
    de = ea.shape[1]
    g = offs.shape[1]
    dout = w2.shape[1]
    grid = e // eb
    const = lambda i: (0, 0)
    return pl.pallas_call(
        _edge_body,
        grid=(grid,),
        in_specs=[
            pl.BlockSpec((eb, hh), lambda i: (i, 0)),
            pl.BlockSpec((eb, 4), lambda i: (i, 0)),
            pl.BlockSpec((eb, de), lambda i: (i, 0)),
            pl.BlockSpec((g, hh), const),
            pl.BlockSpec((de, hh), const),
            pl.BlockSpec((1, hh), const),
            pl.BlockSpec((hh, hh), const),
            pl.BlockSpec((1, hh), const),
            pl.BlockSpec((hh, dout), const),
            pl.BlockSpec((1, dout), const),
            pl.BlockSpec((1, dout), const),
            pl.BlockSpec((1, 1), const),
            pl.BlockSpec((1, g), const),
            pl.BlockSpec((1, 1), const),
        ],
        out_specs=pl.BlockSpec((eb, 144), lambda i: (i, 0)),
        out_shape=jax.ShapeDtypeStruct((e, 144), jnp.float32),
    )(g1, d4, ea, wsm, wea, b0, w1, b1, w2, b2, cw, cb, offs, s)


def _final_body(x_ref, p_ref, w0a_ref, w0b_ref, b0_ref, w1_ref, b1_ref,
                w2_ref, b2_ref, wo_ref, bo_ref, xu_ref, pu_ref):
    agg = p_ref[0] + p_ref[1]
    m = agg[:, :128]
    num = agg[:, 128:131]
    cnt = agg[:, 131:132]
    pu_ref[...] = num / jnp.maximum(cnt, 1.0)
    h = _lrelu(jnp.dot(x_ref[...], w0a_ref[...], preferred_element_type=jnp.float32)
               + jnp.dot(m, w0b_ref[...], preferred_element_type=jnp.float32)
               + b0_ref[...])
    h = _lrelu(jnp.dot(h, w1_ref[...], preferred_element_type=jnp.float32)
               + b1_ref[...])
    h = _lrelu(jnp.dot(h, w2_ref[...], preferred_element_type=jnp.float32)
               + b2_ref[...])
    xu_ref[...] = (jnp.dot(h, wo_ref[...], preferred_element_type=jnp.float32)
                   + bo_ref[...])


def _node_update(x, parts, w0a, w0b, b0, w1, b1, w2, b2, wo, bo, nb):
    n, d = x.shape
    hh = w0a.shape[1]
    grid = n // nb
    const = lambda i: (0, 0)
    return pl.pallas_call(
        _final_body,
        grid=(grid,),
        in_specs=[
            pl.BlockSpec((nb, d), lambda i: (i, 0)),
            pl.BlockSpec((2, nb, 144), lambda i: (0, i, 0)),
            pl.BlockSpec((d, hh), const),
            pl.BlockSpec((d, hh), const),
            pl.BlockSpec((1, hh), const),
            pl.BlockSpec((hh, hh), const),
            pl.BlockSpec((1, hh), const),
            pl.BlockSpec((hh, d), const),
            pl.BlockSpec((1, d), const),
            pl.BlockSpec((d, d), const),
            pl.BlockSpec((1, d), const),
        ],
        out_specs=[
            pl.BlockSpec((nb, d), lambda i: (i, 0)),
            pl.BlockSpec((nb, 3), lambda i: (i, 0)),
        ],
        out_shape=[
            jax.ShapeDtypeStruct((n, d), jnp.float32),
            jax.ShapeDtypeStruct((n, 3), jnp.float32),
        ],
    )(x, parts, w0a, w0b, b0, w1, b1, w2, b2, wo, bo)


def kernel(x, pos, edge_index, batch, edge_attr, msg_W0, msg_b0, msg_W1,
           msg_b1, msg_W2, msg_b2, coord_W, coord_b, node_W0, node_b0,
           node_W1, node_b1, node_W2, node_b2, out_W, out_b):
    n, d = x.shape
    e = edge_index.shape[1]
    g = msg_W0.shape[0] - 2 * d - edge_attr.shape[1]
    src = edge_index[0]
    dst = edge_index[1]

    a, b = _node_proj(x, msg_W0[:d], msg_W0[d:2 * d], nb=2000)

    g1 = a[dst] + b[src]
    pos4 = jnp.pad(pos, ((0, 0), (0, 1)))
    d4 = pos4[dst] - pos4[src]

    step = 16.0 / (g - 1)
    s = jnp.sqrt(0.5) / step
    offs = (jnp.linspace(0.0, 16.0, g) * s).reshape(1, g).astype(jnp.float32)
    s_arr = jnp.full((1, 1), s, jnp.float32)

    packed = _edge_mlp(
        g1, d4, edge_attr,
        msg_W0[2 * d:2 * d + g], msg_W0[2 * d + g:], msg_b0.reshape(1, -1),
        msg_W1, msg_b1.reshape(1, -1), msg_W2, msg_b2.reshape(1, -1),
        coord_W.reshape(1, d), coord_b.reshape(1, 1), offs, s_arr, eb=2000)

    agg = jax.ops.segment_sum(packed, dst, num_segments=n)
    parts = jnp.stack([agg, jnp.zeros_like(agg)])

    x_update, pos_update = _node_update(
        x, parts, node_W0[:d], node_W0[d:], node_b0.reshape(1, -1),
        node_W1, node_b1.reshape(1, -1), node_W2, node_b2.reshape(1, -1),
        out_W, out_b.reshape(1, -1), nb=2000)
    return (x_update, pos_update)


# speedup vs baseline: 3.2273x; 2.6142x over previous
import functools

import jax
import jax.numpy as jnp
from jax import lax
from jax.experimental import pallas as pl
from jax.experimental.pallas import tpu as pltpu
from jax.experimental.pallas import tpu_sc as plsc

_NC = 2
_NS = 16


def _lrelu(v):
    return jnp.maximum(v, 0.2 * v)


def _proj_body(x_ref, p4_ref, wd_ref, ws_ref, t_ref):
    x = x_ref[...]
    p4 = p4_ref[...]
    nb = x.shape[0]
    t_ref[...] = jnp.concatenate(
        [jnp.dot(x, wd_ref[...], preferred_element_type=jnp.float32),
         jnp.dot(x, ws_ref[...], preferred_element_type=jnp.float32),
         p4, jnp.zeros((nb, 60), jnp.float32)], axis=1)


def _node_table(x, pos4, wd, ws, nb):
    n, d = x.shape
    h = wd.shape[1]
    return pl.pallas_call(
        _proj_body,
        grid=(n // nb,),
        in_specs=[
            pl.BlockSpec((nb, d), lambda i: (i, 0)),
            pl.BlockSpec((nb, 4), lambda i: (i, 0)),
            pl.BlockSpec((d, h), lambda i: (0, 0)),
            pl.BlockSpec((d, h), lambda i: (0, 0)),
        ],
        out_specs=pl.BlockSpec((nb, 128), lambda i: (i, 0)),
        out_shape=jax.ShapeDtypeStruct((n, 128), jnp.float32),
    )(x, pos4, wd, ws)


def _sc_gather(tbl, src, dst):
    e = src.shape[0]
    k = 80
    per_w = e // (_NC * _NS)
    chunks = per_w // k
    mesh = plsc.VectorSubcoreMesh(
        core_axis_name="c", subcore_axis_name="s",
        num_cores=_NC, num_subcores=_NS)

    @functools.partial(
        pl.kernel,
        out_type=jax.ShapeDtypeStruct((e, 128), jnp.float32),
        mesh=mesh,
        scratch_types=[
            pltpu.VMEM((k,), jnp.int32),
            pltpu.VMEM((k,), jnp.int32),
            pltpu.VMEM((k, 128), jnp.float32),
            pltpu.VMEM((k, 128), jnp.float32),
            pltpu.VMEM((k, 128), jnp.float32),
            pltpu.SemaphoreType.DMA,
            pltpu.SemaphoreType.DMA,
        ],
    )
    def kern(tbl_hbm, src_hbm, dst_hbm, out_hbm,
             isrc, idst, bufd, bufs, bufo, semd, sems):
        wid = lax.axis_index("s") * _NC + lax.axis_index("c")
        base = wid * per_w
        zero = jnp.zeros((16,), jnp.float32)

        def zrow(r, carry):
            for cc in range(5):
                bufo[r, pl.ds(48 + cc * 16, 16)] = zero
            return carry

        lax.fori_loop(0, k, zrow, 0)

        def body(j, carry):
            off = base + j * k
            pltpu.sync_copy(dst_hbm.at[pl.ds(off, k)], idst)
            pltpu.sync_copy(src_hbm.at[pl.ds(off, k)], isrc)
            cd = pltpu.async_copy(tbl_hbm.at[idst], bufd, semd)
            cs = pltpu.async_copy(tbl_hbm.at[isrc], bufs, sems)
            cd.wait()
            cs.wait()

            def row(r, c2):
                bufo[r, pl.ds(0, 16)] = (bufd[r, pl.ds(0, 16)]
                                         + bufs[r, pl.ds(32, 16)])
                bufo[r, pl.ds(16, 16)] = (bufd[r, pl.ds(16, 16)]
                                          + bufs[r, pl.ds(48, 16)])
                bufo[r, pl.ds(32, 16)] = (bufd[r, pl.ds(64, 16)]
                                          - bufs[r, pl.ds(64, 16)])
                return c2

            lax.fori_loop(0, k, row, 0)
            pltpu.sync_copy(bufo, out_hbm.at[pl.ds(off, k)])
            return carry

        lax.fori_loop(0, chunks, body, 0)

    return kern(tbl, src, dst)


def _edge_body(gp_ref, ea_ref, wsm_ref, wea_ref, b0_ref, w1_ref,
               b1_ref, w2_ref, b2_ref, cw_ref, cb_ref, offs_ref, s_ref,
               msg_ref, wv_ref):
    gp = gp_ref[...]
    d4 = gp[:, 32:36]
    dd = jnp.sum(d4 * d4, axis=1, keepdims=True)
    dist = jnp.sqrt(dd)
    s = s_ref[0, 0]
    t = dist * s - offs_ref[...]
    sm = jnp.exp(-(t * t))
    pre0 = (gp[:, :32]
            + jnp.dot(sm, wsm_ref[...], preferred_element_type=jnp.float32)
            + jnp.dot(ea_ref[...], wea_ref[...], preferred_element_type=jnp.float32)
            + b0_ref[...])
    h1 = _lrelu(pre0)
    h2 = _lrelu(jnp.dot(h1, w1_ref[...], preferred_element_type=jnp.float32)
                + b1_ref[...])
    z = jnp.dot(h2, w2_ref[...], preferred_element_type=jnp.float32) + b2_ref[...]
    msg = _lrelu(z)
    msg_ref[...] = msg
    ew = jnp.sum(msg * cw_ref[...], axis=1, keepdims=True) + cb_ref[0, 0]
    inv = ew / (dist + 1e-6)
    wv = d4[:, :3] * inv
    eb = msg.shape[0]
    wv_ref[...] = jnp.concatenate(
        [wv, jnp.ones((eb, 1), jnp.float32),
         jnp.zeros((eb, 124), jnp.float32)], axis=1)


def _edge_mlp(gp, ea, wsm, wea, b0, w1, b1, w2, b2, cw, cb, offs, s, eb):
    e = gp.shape[0]
    de = ea.shape[1]
    g = offs.shape[1]
    dout = w2.shape[1]
    const = lambda i: (0, 0)
    return pl.pallas_call(
        _edge_body,
        grid=(e // eb,),
        in_specs=[
            pl.BlockSpec((eb, 128), lambda i: (i, 0)),
            pl.BlockSpec((eb, de), lambda i: (i, 0)),
            pl.BlockSpec((g, 32), const),
            pl.BlockSpec((de, 32), const),
            pl.BlockSpec((1, 32), const),
            pl.BlockSpec((32, 32), const),
            pl.BlockSpec((1, 32), const),
            pl.BlockSpec((32, dout), const),
            pl.BlockSpec((1, dout), const),
            pl.BlockSpec((1, dout), const),
            pl.BlockSpec((1, 1), const),
            pl.BlockSpec((1, g), const),
            pl.BlockSpec((1, 1), const),
        ],
        out_specs=[
            pl.BlockSpec((eb, 128), lambda i: (i, 0)),
            pl.BlockSpec((eb, 128), lambda i: (i, 0)),
        ],
        out_shape=[
            jax.ShapeDtypeStruct((e, 128), jnp.float32),
            jax.ShapeDtypeStruct((e, 128), jnp.float32),
        ],
    )(gp, ea, wsm, wea, b0, w1, b1, w2, b2, cw, cb, offs, s)


def _sc_scatter(rows_in, dst, n):
    e = rows_in.shape[0]
    k = 80
    per_w = e // (_NC * _NS)
    chunks = per_w // k
    zrows = 80
    nblk = n // zrows
    mesh = plsc.VectorSubcoreMesh(
        core_axis_name="c", subcore_axis_name="s",
        num_cores=_NC, num_subcores=_NS)

    @functools.partial(
        pl.kernel,
        out_type=jax.ShapeDtypeStruct((_NC, n, 128), jnp.float32),
        mesh=mesh,
        scratch_types=[
            pltpu.VMEM((k,), jnp.int32),
            pltpu.VMEM((k, 128), jnp.float32),
            pltpu.VMEM((zrows, 128), jnp.float32),
            pltpu.VMEM_SHARED((n, 128), jnp.float32),
        ],
    )
    def kern(rows_hbm, dst_hbm, out_hbm, idx, rows, zbuf, acc):
        c = lax.axis_index("c")
        s = lax.axis_index("s")
        wid = s * _NC + c
        zero = jnp.zeros((16,), jnp.float32)

        def zr(r, carry):
            for cc in range(8):
                zbuf[r, pl.ds(cc * 16, 16)] = zero
            return carry

        lax.fori_loop(0, zrows, zr, 0)

        def zc(i, carry):
            blk = i * _NS + s

            @pl.when(blk < nblk)
            def _():
                pltpu.sync_copy(zbuf, acc.at[pl.ds(blk * zrows, zrows)])

            return carry

        lax.fori_loop(0, (nblk + _NS - 1) // _NS, zc, 0)
        plsc.subcore_barrier()

        base = wid * per_w

        def body(j, carry):
            off = base + j * k
            pltpu.sync_copy(dst_hbm.at[pl.ds(off, k)], idx)
            pltpu.sync_copy(rows_hbm.at[pl.ds(off, k)], rows)
            pltpu.sync_copy(rows, acc.at[idx], add=True)
            return carry

        lax.fori_loop(0, chunks, body, 0)
        plsc.subcore_barrier()

        def wb(i, carry):
            blk = i * _NS + s

            @pl.when(blk < nblk)
            def _():
                pltpu.sync_copy(acc.at[pl.ds(blk * zrows, zrows)],
                                out_hbm.at[c].at[pl.ds(blk * zrows, zrows)])

            return carry

        lax.fori_loop(0, (nblk + _NS - 1) // _NS, wb, 0)

    return kern(rows_in, dst)


def _final_body(x_ref, pm_ref, pw_ref, w0a_ref, w0b_ref, b0_ref, w1_ref,
                b1_ref, w2_ref, b2_ref, wo_ref, bo_ref, xu_ref, pu_ref):
    m = pm_ref[0] + pm_ref[1]
    aw = pw_ref[0] + pw_ref[1]
    num = aw[:, 0:3]
    cnt = aw[:, 3:4]
    pu_ref[...] = num / jnp.maximum(cnt, 1.0)
    h = _lrelu(jnp.dot(x_ref[...], w0a_ref[...], preferred_element_type=jnp.float32)
               + jnp.dot(m, w0b_ref[...], preferred_element_type=jnp.float32)
               + b0_ref[...])
    h = _lrelu(jnp.dot(h, w1_ref[...], preferred_element_type=jnp.float32)
               + b1_ref[...])
    h = _lrelu(jnp.dot(h, w2_ref[...], preferred_element_type=jnp.float32)
               + b2_ref[...])
    xu_ref[...] = (jnp.dot(h, wo_ref[...], preferred_element_type=jnp.float32)
                   + bo_ref[...])


def _node_update(x, pm, pw, w0a, w0b, b0, w1, b1, w2, b2, wo, bo, nb):
    n, d = x.shape
    hh = w0a.shape[1]
    const = lambda i: (0, 0)
    return pl.pallas_call(
        _final_body,
        grid=(n // nb,),
        in_specs=[
            pl.BlockSpec((nb, d), lambda i: (i, 0)),
            pl.BlockSpec((2, nb, 128), lambda i: (0, i, 0)),
            pl.BlockSpec((2, nb, 128), lambda i: (0, i, 0)),
            pl.BlockSpec((d, hh), const),
            pl.BlockSpec((d, hh), const),
            pl.BlockSpec((1, hh), const),
            pl.BlockSpec((hh, hh), const),
            pl.BlockSpec((1, hh), const),
            pl.BlockSpec((hh, d), const),
            pl.BlockSpec((1, d), const),
            pl.BlockSpec((d, d), const),
            pl.BlockSpec((1, d), const),
        ],
        out_specs=[
            pl.BlockSpec((nb, d), lambda i: (i, 0)),
            pl.BlockSpec((nb, 3), lambda i: (i, 0)),
        ],
        out_shape=[
            jax.ShapeDtypeStruct((n, d), jnp.float32),
            jax.ShapeDtypeStruct((n, 3), jnp.float32),
        ],
    )(x, pm, pw, w0a, w0b, b0, w1, b1, w2, b2, wo, bo)


def kernel(x, pos, edge_index, batch, edge_attr, msg_W0, msg_b0, msg_W1,
           msg_b1, msg_W2, msg_b2, coord_W, coord_b, node_W0, node_b0,
           node_W1, node_b1, node_W2, node_b2, out_W, out_b):
    n, d = x.shape
    g = msg_W0.shape[0] - 2 * d - edge_attr.shape[1]
    src = edge_index[0]
    dst = edge_index[1]

    pos4 = jnp.pad(pos, ((0, 0), (0, 1)))
    tbl = _node_table(x, pos4, msg_W0[:d], msg_W0[d:2 * d], nb=2000)

    gp = _sc_gather(tbl, src, dst)

    step = 16.0 / (g - 1)
    s = (0.5 ** 0.5) / step
    offs = (jnp.linspace(0.0, 16.0, g) * s).reshape(1, g).astype(jnp.float32)
    s_arr = jnp.full((1, 1), s, jnp.float32)

    msg, wv = _edge_mlp(
        gp, edge_attr,
        msg_W0[2 * d:2 * d + g], msg_W0[2 * d + g:], msg_b0.reshape(1, -1),
        msg_W1, msg_b1.reshape(1, -1), msg_W2, msg_b2.reshape(1, -1),
        coord_W.reshape(1, d), coord_b.reshape(1, 1), offs, s_arr, eb=2000)

    pm = _sc_scatter(msg, dst, n)
    pw = _sc_scatter(wv, dst, n)

    x_update, pos_update = _node_update(
        x, pm, pw, node_W0[:d], node_W0[d:], node_b0.reshape(1, -1),
        node_W1, node_b1.reshape(1, -1), node_W2, node_b2.reshape(1, -1),
        out_W, out_b.reshape(1, -1), nb=2000)
    return (x_update, pos_update)


# speedup vs baseline: 5.0173x; 1.5547x over previous
import functools

import jax
import jax.numpy as jnp
from jax import lax
from jax.experimental import pallas as pl
from jax.experimental.pallas import tpu as pltpu
from jax.experimental.pallas import tpu_sc as plsc

_NC = 2
_NS = 16


def _lrelu(v):
    return jnp.where(v >= 0, v, 0.2 * v)


def _proj_body(x_ref, p4_ref, wd_ref, ws_ref, t_ref):
    x = x_ref[...]
    p4 = p4_ref[...]
    nb = x.shape[0]
    t_ref[...] = jnp.concatenate(
        [jnp.dot(x, wd_ref[...], preferred_element_type=jnp.float32),
         jnp.dot(x, ws_ref[...], preferred_element_type=jnp.float32),
         p4, jnp.zeros((nb, 60), jnp.float32)], axis=1)


def _node_table(x, pos4, wd, ws, nb):
    n, d = x.shape
    h = wd.shape[1]
    return pl.pallas_call(
        _proj_body,
        grid=(n // nb,),
        in_specs=[
            pl.BlockSpec((nb, d), lambda i: (i, 0)),
            pl.BlockSpec((nb, 4), lambda i: (i, 0)),
            pl.BlockSpec((d, h), lambda i: (0, 0)),
            pl.BlockSpec((d, h), lambda i: (0, 0)),
        ],
        out_specs=pl.BlockSpec((nb, 128), lambda i: (i, 0)),
        out_shape=jax.ShapeDtypeStruct((n, 128), jnp.float32),
    )(x, pos4, wd, ws)


def _sc_gather(tbl, src, dst):
    e = src.shape[0]
    k = 80
    per_w = e // (_NC * _NS)
    chunks = per_w // k
    mesh = plsc.VectorSubcoreMesh(
        core_axis_name="c", subcore_axis_name="s",
        num_cores=_NC, num_subcores=_NS)

    @functools.partial(
        pl.kernel,
        out_type=jax.ShapeDtypeStruct((e, 128), jnp.float32),
        mesh=mesh,
        scratch_types=[
            pltpu.VMEM((k,), jnp.int32),
            pltpu.VMEM((k,), jnp.int32),
            pltpu.VMEM((k, 128), jnp.float32),
            pltpu.VMEM((k, 128), jnp.float32),
            pltpu.VMEM((k, 128), jnp.float32),
            pltpu.SemaphoreType.DMA,
            pltpu.SemaphoreType.DMA,
        ],
    )
    def kern(tbl_hbm, src_hbm, dst_hbm, out_hbm,
             isrc, idst, bufd, bufs, bufo, semd, sems):
        wid = lax.axis_index("s") * _NC + lax.axis_index("c")
        base = wid * per_w
        zero = jnp.zeros((16,), jnp.float32)

        def zrow(r, carry):
            for cc in range(5):
                bufo[r, pl.ds(48 + cc * 16, 16)] = zero
            return carry

        lax.fori_loop(0, k, zrow, 0)

        def body(j, carry):
            off = base + j * k
            pltpu.sync_copy(dst_hbm.at[pl.ds(off, k)], idst)
            pltpu.sync_copy(src_hbm.at[pl.ds(off, k)], isrc)
            cd = pltpu.async_copy(tbl_hbm.at[idst], bufd, semd)
            cs = pltpu.async_copy(tbl_hbm.at[isrc], bufs, sems)
            cd.wait()
            cs.wait()

            def row(r, c2):
                bufo[r, pl.ds(0, 16)] = (bufd[r, pl.ds(0, 16)]
                                         + bufs[r, pl.ds(32, 16)])
                bufo[r, pl.ds(16, 16)] = (bufd[r, pl.ds(16, 16)]
                                          + bufs[r, pl.ds(48, 16)])
                bufo[r, pl.ds(32, 16)] = (bufd[r, pl.ds(64, 16)]
                                          - bufs[r, pl.ds(64, 16)])
                return c2

            lax.fori_loop(0, k, row, 0)
            pltpu.sync_copy(bufo, out_hbm.at[pl.ds(off, k)])
            return carry

        lax.fori_loop(0, chunks, body, 0)

    return kern(tbl, src, dst)


def _edge_body(gp_ref, ea_ref, wsm_ref, wea_ref, b0_ref, w1_ref,
               b1_ref, w2_ref, b2_ref, cw_ref, cb_ref, offs_ref, s_ref,
               ones4_ref, w0b_ref, out_ref):
    gp = gp_ref[...]
    d4 = gp[:, 32:36]
    dd32 = jnp.dot(d4 * d4, ones4_ref[...],
                   preferred_element_type=jnp.float32)
    dist32 = jnp.sqrt(dd32)
    s = s_ref[0, 0]
    t = dist32 * s - offs_ref[...]
    sm = jnp.exp(-(t * t))
    pre0 = (gp[:, :32]
            + jnp.dot(sm, wsm_ref[...], preferred_element_type=jnp.float32)
            + jnp.dot(ea_ref[...], wea_ref[...], preferred_element_type=jnp.float32)
            + b0_ref[...])
    h1 = _lrelu(pre0)
    h2 = _lrelu(jnp.dot(h1, w1_ref[...], preferred_element_type=jnp.float32)
                + b1_ref[...])
    z = jnp.dot(h2, w2_ref[...], preferred_element_type=jnp.float32) + b2_ref[...]
    msg = _lrelu(z)
    m32 = jnp.dot(msg, w0b_ref[...], preferred_element_type=jnp.float32)
    ew = jnp.dot(msg, cw_ref[...], preferred_element_type=jnp.float32) \
        + cb_ref[0, 0]
    q = ew / (dist32[:, 0:1] + 1e-6)
    eb = msg.shape[0]
    w4 = jnp.concatenate(
        [d4[:, :3] * q, jnp.ones((eb, 1), jnp.float32)], axis=1)
    out_ref[...] = jnp.concatenate(
        [m32, w4, jnp.zeros((eb, 92), jnp.float32)], axis=1)


def _edge_mlp(gp, ea, wsm, wea, b0, w1, b1, w2, b2, cw, cb, offs, s,
              ones4, w0b, eb):
    e = gp.shape[0]
    de = ea.shape[1]
    g = offs.shape[1]
    dout = w2.shape[1]
    const = lambda i: (0, 0)
    return pl.pallas_call(
        _edge_body,
        grid=(e // eb,),
        in_specs=[
            pl.BlockSpec((eb, 128), lambda i: (i, 0)),
            pl.BlockSpec((eb, de), lambda i: (i, 0)),
            pl.BlockSpec((g, 32), const),
            pl.BlockSpec((de, 32), const),
            pl.BlockSpec((1, 32), const),
            pl.BlockSpec((32, 32), const),
            pl.BlockSpec((1, 32), const),
            pl.BlockSpec((32, dout), const),
            pl.BlockSpec((1, dout), const),
            pl.BlockSpec((dout, 1), const),
            pl.BlockSpec((1, 1), const),
            pl.BlockSpec((1, g), const),
            pl.BlockSpec((1, 1), const),
            pl.BlockSpec((4, g), const),
            pl.BlockSpec((dout, 32), const),
        ],
        out_specs=pl.BlockSpec((eb, 128), lambda i: (i, 0)),
        out_shape=jax.ShapeDtypeStruct((e, 128), jnp.float32),
    )(gp, ea, wsm, wea, b0, w1, b1, w2, b2, cw, cb, offs, s, ones4, w0b)


def _sc_scatter(rows_in, dst, n):
    e = rows_in.shape[0]
    k = 80
    per_w = e // (_NC * _NS)
    chunks = per_w // k
    zrows = 80
    nblk = n // zrows
    mesh = plsc.VectorSubcoreMesh(
        core_axis_name="c", subcore_axis_name="s",
        num_cores=_NC, num_subcores=_NS)

    @functools.partial(
        pl.kernel,
        out_type=jax.ShapeDtypeStruct((_NC, n, 128), jnp.float32),
        mesh=mesh,
        scratch_types=[
            pltpu.VMEM((k,), jnp.int32),
            pltpu.VMEM((k, 128), jnp.float32),
            pltpu.VMEM((zrows, 128), jnp.float32),
            pltpu.VMEM_SHARED((n, 128), jnp.float32),
        ],
    )
    def kern(rows_hbm, dst_hbm, out_hbm, idx, rows, zbuf, acc):
        c = lax.axis_index("c")
        s = lax.axis_index("s")
        wid = s * _NC + c
        zero = jnp.zeros((16,), jnp.float32)

        def zr(r, carry):
            for cc in range(8):
                zbuf[r, pl.ds(cc * 16, 16)] = zero
            return carry

        lax.fori_loop(0, zrows, zr, 0)

        def zc(i, carry):
            blk = i * _NS + s

            @pl.when(blk < nblk)
            def _():
                pltpu.sync_copy(zbuf, acc.at[pl.ds(blk * zrows, zrows)])

            return carry

        lax.fori_loop(0, (nblk + _NS - 1) // _NS, zc, 0)
        plsc.subcore_barrier()

        base = wid * per_w

        def body(j, carry):
            off = base + j * k
            pltpu.sync_copy(dst_hbm.at[pl.ds(off, k)], idx)
            pltpu.sync_copy(rows_hbm.at[pl.ds(off, k)], rows)
            pltpu.sync_copy(rows, acc.at[idx], add=True)
            return carry

        lax.fori_loop(0, chunks, body, 0)
        plsc.subcore_barrier()

        def wb(i, carry):
            blk = i * _NS + s

            @pl.when(blk < nblk)
            def _():
                pltpu.sync_copy(acc.at[pl.ds(blk * zrows, zrows)],
                                out_hbm.at[c].at[pl.ds(blk * zrows, zrows)])

            return carry

        lax.fori_loop(0, (nblk + _NS - 1) // _NS, wb, 0)

    return kern(rows_in, dst)


def _final_body(x_ref, pm_ref, w0a_ref, b0_ref, w1_ref,
                b1_ref, w2_ref, b2_ref, wo_ref, bo_ref, xu_ref, pu_ref):
    agg = pm_ref[0] + pm_ref[1]
    m32 = agg[:, 0:32]
    num = agg[:, 32:35]
    cnt = agg[:, 35:36]
    pu_ref[...] = num / jnp.maximum(cnt, 1.0)
    h = _lrelu(jnp.dot(x_ref[...], w0a_ref[...], preferred_element_type=jnp.float32)
               + m32 + b0_ref[...])
    h = _lrelu(jnp.dot(h, w1_ref[...], preferred_element_type=jnp.float32)
               + b1_ref[...])
    h = _lrelu(jnp.dot(h, w2_ref[...], preferred_element_type=jnp.float32)
               + b2_ref[...])
    xu_ref[...] = (jnp.dot(h, wo_ref[...], preferred_element_type=jnp.float32)
                   + bo_ref[...])


def _node_update(x, pm, w0a, b0, w1, b1, w2, b2, wo, bo, nb):
    n, d = x.shape
    hh = w0a.shape[1]
    const = lambda i: (0, 0)
    return pl.pallas_call(
        _final_body,
        grid=(n // nb,),
        in_specs=[
            pl.BlockSpec((nb, d), lambda i: (i, 0)),
            pl.BlockSpec((2, nb, 128), lambda i: (0, i, 0)),
            pl.BlockSpec((d, hh), const),
            pl.BlockSpec((1, hh), const),
            pl.BlockSpec((hh, hh), const),
            pl.BlockSpec((1, hh), const),
            pl.BlockSpec((hh, d), const),
            pl.BlockSpec((1, d), const),
            pl.BlockSpec((d, d), const),
            pl.BlockSpec((1, d), const),
        ],
        out_specs=[
            pl.BlockSpec((nb, d), lambda i: (i, 0)),
            pl.BlockSpec((nb, 3), lambda i: (i, 0)),
        ],
        out_shape=[
            jax.ShapeDtypeStruct((n, d), jnp.float32),
            jax.ShapeDtypeStruct((n, 3), jnp.float32),
        ],
    )(x, pm, w0a, b0, w1, b1, w2, b2, wo, bo)


def kernel(x, pos, edge_index, batch, edge_attr, msg_W0, msg_b0, msg_W1,
           msg_b1, msg_W2, msg_b2, coord_W, coord_b, node_W0, node_b0,
           node_W1, node_b1, node_W2, node_b2, out_W, out_b):
    n, d = x.shape
    g = msg_W0.shape[0] - 2 * d - edge_attr.shape[1]
    src = edge_index[0]
    dst = edge_index[1]

    pos4 = jnp.pad(pos, ((0, 0), (0, 1)))
    tbl = _node_table(x, pos4, msg_W0[:d], msg_W0[d:2 * d], nb=2000)

    gp = _sc_gather(tbl, src, dst)

    step = 16.0 / (g - 1)
    s = (0.5 ** 0.5) / step
    offs = (jnp.linspace(0.0, 16.0, g) * s).reshape(1, g).astype(jnp.float32)
    s_arr = jnp.full((1, 1), s, jnp.float32)

    ones4 = jnp.ones((4, g), jnp.float32)
    packed = _edge_mlp(
        gp, edge_attr,
        msg_W0[2 * d:2 * d + g], msg_W0[2 * d + g:], msg_b0.reshape(1, -1),
        msg_W1, msg_b1.reshape(1, -1), msg_W2, msg_b2.reshape(1, -1),
        coord_W.reshape(d, 1), coord_b.reshape(1, 1), offs, s_arr,
        ones4, node_W0[d:], eb=4000)

    pm = _sc_scatter(packed, dst, n)

    x_update, pos_update = _node_update(
        x, pm, node_W0[:d], node_b0.reshape(1, -1),
        node_W1, node_b1.reshape(1, -1), node_W2, node_b2.reshape(1, -1),
        out_W, out_b.reshape(1, -1), nb=2000)
    return (x_update, pos_update)


# speedup vs baseline: 5.8832x; 1.1726x over previous
import functools

import jax
import jax.numpy as jnp
from jax import lax
from jax.experimental import pallas as pl
from jax.experimental.pallas import tpu as pltpu
from jax.experimental.pallas import tpu_sc as plsc

_NC = 2
_NS = 16


def _lrelu(v):
    return jnp.where(v >= 0, v, 0.2 * v)


def _proj_body(x_ref, p4_ref, wd_ref, ws_ref, t_ref):
    x = x_ref[...]
    p4 = p4_ref[...]
    nb = x.shape[0]
    t_ref[...] = jnp.concatenate(
        [jnp.dot(x, wd_ref[...], preferred_element_type=jnp.float32),
         jnp.dot(x, ws_ref[...], preferred_element_type=jnp.float32),
         p4, jnp.zeros((nb, 60), jnp.float32)], axis=1)


def _node_table(x, pos4, wd, ws, nb):
    n, d = x.shape
    h = wd.shape[1]
    return pl.pallas_call(
        _proj_body,
        grid=(n // nb,),
        in_specs=[
            pl.BlockSpec((nb, d), lambda i: (i, 0)),
            pl.BlockSpec((nb, 4), lambda i: (i, 0)),
            pl.BlockSpec((d, h), lambda i: (0, 0)),
            pl.BlockSpec((d, h), lambda i: (0, 0)),
        ],
        out_specs=pl.BlockSpec((nb, 128), lambda i: (i, 0)),
        out_shape=jax.ShapeDtypeStruct((n, 128), jnp.float32),
    )(x, pos4, wd, ws)


def _sc_gather(tbl, src, dst):
    e = src.shape[0]
    k = 80
    per_w = e // (_NC * _NS)
    chunks = per_w // k
    mesh = plsc.VectorSubcoreMesh(
        core_axis_name="c", subcore_axis_name="s",
        num_cores=_NC, num_subcores=_NS)

    @functools.partial(
        pl.kernel,
        out_type=jax.ShapeDtypeStruct((e, 128), jnp.float32),
        mesh=mesh,
        scratch_types=[
            pltpu.VMEM((k,), jnp.int32),
            pltpu.VMEM((k,), jnp.int32),
            pltpu.VMEM((k, 128), jnp.float32),
            pltpu.VMEM((k, 128), jnp.float32),
            pltpu.VMEM((k, 128), jnp.float32),
            pltpu.SemaphoreType.DMA,
            pltpu.SemaphoreType.DMA,
        ],
    )
    def kern(tbl_hbm, src_hbm, dst_hbm, out_hbm,
             isrc, idst, bufd, bufs, bufo, semd, sems):
        wid = lax.axis_index("s") * _NC + lax.axis_index("c")
        base = wid * per_w
        zero = jnp.zeros((16,), jnp.float32)

        def zrow(r, carry):
            for cc in range(5):
                bufo[r, pl.ds(48 + cc * 16, 16)] = zero
            return carry

        lax.fori_loop(0, k, zrow, 0)

        def body(j, carry):
            off = base + j * k
            pltpu.sync_copy(dst_hbm.at[pl.ds(off, k)], idst)
            pltpu.sync_copy(src_hbm.at[pl.ds(off, k)], isrc)
            cd = pltpu.async_copy(tbl_hbm.at[idst], bufd, semd)
            cs = pltpu.async_copy(tbl_hbm.at[isrc], bufs, sems)
            cd.wait()
            cs.wait()

            def row(r, c2):
                bufo[r, pl.ds(0, 16)] = (bufd[r, pl.ds(0, 16)]
                                         + bufs[r, pl.ds(32, 16)])
                bufo[r, pl.ds(16, 16)] = (bufd[r, pl.ds(16, 16)]
                                          + bufs[r, pl.ds(48, 16)])
                bufo[r, pl.ds(32, 16)] = (bufd[r, pl.ds(64, 16)]
                                          - bufs[r, pl.ds(64, 16)])
                return c2

            lax.fori_loop(0, k, row, 0)
            pltpu.sync_copy(bufo, out_hbm.at[pl.ds(off, k)])
            return carry

        lax.fori_loop(0, chunks, body, 0)

    return kern(tbl, src, dst)


def _edge_body(gp_ref, ea_ref, wsm_ref, wea_ref, b0_ref, w1_ref,
               b1_ref, w2_ref, b2_ref, cw_ref, cb_ref, offs_ref, s_ref,
               ones4_ref, w0b_ref, out_ref):
    gp = gp_ref[...]
    d4 = gp[:, 32:36]
    dd32 = jnp.dot(d4 * d4, ones4_ref[...],
                   preferred_element_type=jnp.float32)
    dist32 = jnp.sqrt(dd32)
    s = s_ref[0, 0]
    t = dist32 * s - offs_ref[...]
    sm = jnp.exp(-(t * t))
    pre0 = (gp[:, :32]
            + jnp.dot(sm, wsm_ref[...], preferred_element_type=jnp.float32)
            + jnp.dot(ea_ref[...], wea_ref[...], preferred_element_type=jnp.float32)
            + b0_ref[...])
    h1 = _lrelu(pre0)
    h2 = _lrelu(jnp.dot(h1, w1_ref[...], preferred_element_type=jnp.float32)
                + b1_ref[...])
    z = jnp.dot(h2, w2_ref[...], preferred_element_type=jnp.float32) + b2_ref[...]
    msg = _lrelu(z)
    m32 = jnp.dot(msg, w0b_ref[...], preferred_element_type=jnp.float32)
    ew = jnp.dot(msg, cw_ref[...], preferred_element_type=jnp.float32) \
        + cb_ref[0, 0]
    q = ew / (dist32[:, 0:1] + 1e-6)
    eb = msg.shape[0]
    w4 = jnp.concatenate(
        [d4[:, :3] * q, jnp.ones((eb, 1), jnp.float32)], axis=1)
    out_ref[...] = jnp.concatenate(
        [m32, w4, jnp.zeros((eb, 92), jnp.float32)], axis=1)


def _edge_mlp(gp, ea, wsm, wea, b0, w1, b1, w2, b2, cw, cb, offs, s,
              ones4, w0b, eb):
    e = gp.shape[0]
    de = ea.shape[1]
    g = offs.shape[1]
    dout = w2.shape[1]
    const = lambda i: (0, 0)
    return pl.pallas_call(
        _edge_body,
        grid=(e // eb,),
        in_specs=[
            pl.BlockSpec((eb, 128), lambda i: (i, 0)),
            pl.BlockSpec((eb, de), lambda i: (i, 0)),
            pl.BlockSpec((g, 32), const),
            pl.BlockSpec((de, 32), const),
            pl.BlockSpec((1, 32), const),
            pl.BlockSpec((32, 32), const),
            pl.BlockSpec((1, 32), const),
            pl.BlockSpec((32, dout), const),
            pl.BlockSpec((1, dout), const),
            pl.BlockSpec((dout, 1), const),
            pl.BlockSpec((1, 1), const),
            pl.BlockSpec((1, g), const),
            pl.BlockSpec((1, 1), const),
            pl.BlockSpec((4, g), const),
            pl.BlockSpec((dout, 32), const),
        ],
        out_specs=pl.BlockSpec((eb, 128), lambda i: (i, 0)),
        out_shape=jax.ShapeDtypeStruct((e, 128), jnp.float32),
    )(gp, ea, wsm, wea, b0, w1, b1, w2, b2, cw, cb, offs, s, ones4, w0b)


def _sc_scatter(rows_in, dst, n):
    e = rows_in.shape[0]
    k = 80
    per_w = e // (_NC * _NS)
    chunks = per_w // k
    zrows = 80
    nblk = n // zrows
    mesh = plsc.VectorSubcoreMesh(
        core_axis_name="c", subcore_axis_name="s",
        num_cores=_NC, num_subcores=_NS)

    nbuf = 3

    @functools.partial(
        pl.kernel,
        out_type=jax.ShapeDtypeStruct((_NC, n, 128), jnp.float32),
        mesh=mesh,
        scratch_types=[
            pltpu.VMEM((nbuf, k), jnp.int32),
            pltpu.VMEM((nbuf, k, 128), jnp.float32),
            pltpu.VMEM((zrows, 128), jnp.float32),
            pltpu.VMEM_SHARED((n, 128), jnp.float32),
            pltpu.SemaphoreType.DMA((nbuf,)),
            pltpu.SemaphoreType.DMA((nbuf,)),
        ],
    )
    def kern(rows_hbm, dst_hbm, out_hbm, idx, rows, zbuf, acc, fsem, ssem):
        c = lax.axis_index("c")
        s = lax.axis_index("s")
        wid = s * _NC + c
        zero = jnp.zeros((16,), jnp.float32)

        def zr(r, carry):
            for cc in range(8):
                zbuf[r, pl.ds(cc * 16, 16)] = zero
            return carry

        lax.fori_loop(0, zrows, zr, 0)

        def zc(i, carry):
            blk = i * _NS + s

            @pl.when(blk < nblk)
            def _():
                pltpu.sync_copy(zbuf, acc.at[pl.ds(blk * zrows, zrows)])

            return carry

        lax.fori_loop(0, (nblk + _NS - 1) // _NS, zc, 0)
        plsc.subcore_barrier()

        base = wid * per_w

        def fetch(j):
            p = lax.rem(j, nbuf)
            off = base + j * k
            pltpu.async_copy(dst_hbm.at[pl.ds(off, k)], idx.at[p], fsem.at[p])
            pltpu.async_copy(rows_hbm.at[pl.ds(off, k)], rows.at[p],
                             fsem.at[p])

        def wait_fetch(j):
            p = lax.rem(j, nbuf)
            pltpu.make_async_copy(dst_hbm.at[pl.ds(0, k)], idx.at[p],
                                  fsem.at[p]).wait()
            pltpu.make_async_copy(rows_hbm.at[pl.ds(0, k)], rows.at[p],
                                  fsem.at[p]).wait()

        def scat(j):
            p = lax.rem(j, nbuf)
            pltpu.async_copy(rows.at[p], acc.at[idx.at[p]], ssem.at[p],
                             add=True)

        def wait_scat(j):
            p = lax.rem(j, nbuf)
            pltpu.make_async_copy(rows.at[p], acc.at[idx.at[p]],
                                  ssem.at[p]).wait()

        fetch(0)
        fetch(1)

        def body(j, carry):
            @pl.when(j >= 1)
            def _():
                wait_scat(j - 1)

            @pl.when(j + 2 < chunks)
            def _():
                fetch(j + 2)

            wait_fetch(j)
            scat(j)
            return carry

        lax.fori_loop(0, chunks, body, 0)
        wait_scat(chunks - 1)
        plsc.subcore_barrier()

        def wb(i, carry):
            blk = i * _NS + s

            @pl.when(blk < nblk)
            def _():
                pltpu.sync_copy(acc.at[pl.ds(blk * zrows, zrows)],
                                out_hbm.at[c].at[pl.ds(blk * zrows, zrows)])

            return carry

        lax.fori_loop(0, (nblk + _NS - 1) // _NS, wb, 0)

    return kern(rows_in, dst)


def _final_body(x_ref, pm_ref, w0a_ref, b0_ref, w1_ref,
                b1_ref, w2_ref, b2_ref, wo_ref, bo_ref, xu_ref, pu_ref):
    agg = pm_ref[0] + pm_ref[1]
    m32 = agg[:, 0:32]
    num = agg[:, 32:35]
    cnt = agg[:, 35:36]
    pu_ref[...] = num / jnp.maximum(cnt, 1.0)
    h = _lrelu(jnp.dot(x_ref[...], w0a_ref[...], preferred_element_type=jnp.float32)
               + m32 + b0_ref[...])
    h = _lrelu(jnp.dot(h, w1_ref[...], preferred_element_type=jnp.float32)
               + b1_ref[...])
    h = _lrelu(jnp.dot(h, w2_ref[...], preferred_element_type=jnp.float32)
               + b2_ref[...])
    xu_ref[...] = (jnp.dot(h, wo_ref[...], preferred_element_type=jnp.float32)
                   + bo_ref[...])


def _node_update(x, pm, w0a, b0, w1, b1, w2, b2, wo, bo, nb):
    n, d = x.shape
    hh = w0a.shape[1]
    const = lambda i: (0, 0)
    return pl.pallas_call(
        _final_body,
        grid=(n // nb,),
        in_specs=[
            pl.BlockSpec((nb, d), lambda i: (i, 0)),
            pl.BlockSpec((2, nb, 128), lambda i: (0, i, 0)),
            pl.BlockSpec((d, hh), const),
            pl.BlockSpec((1, hh), const),
            pl.BlockSpec((hh, hh), const),
            pl.BlockSpec((1, hh), const),
            pl.BlockSpec((hh, d), const),
            pl.BlockSpec((1, d), const),
            pl.BlockSpec((d, d), const),
            pl.BlockSpec((1, d), const),
        ],
        out_specs=[
            pl.BlockSpec((nb, d), lambda i: (i, 0)),
            pl.BlockSpec((nb, 3), lambda i: (i, 0)),
        ],
        out_shape=[
            jax.ShapeDtypeStruct((n, d), jnp.float32),
            jax.ShapeDtypeStruct((n, 3), jnp.float32),
        ],
    )(x, pm, w0a, b0, w1, b1, w2, b2, wo, bo)


def kernel(x, pos, edge_index, batch, edge_attr, msg_W0, msg_b0, msg_W1,
           msg_b1, msg_W2, msg_b2, coord_W, coord_b, node_W0, node_b0,
           node_W1, node_b1, node_W2, node_b2, out_W, out_b):
    n, d = x.shape
    g = msg_W0.shape[0] - 2 * d - edge_attr.shape[1]
    src = edge_index[0]
    dst = edge_index[1]

    pos4 = jnp.pad(pos, ((0, 0), (0, 1)))
    tbl = _node_table(x, pos4, msg_W0[:d], msg_W0[d:2 * d], nb=2000)

    gp = _sc_gather(tbl, src, dst)

    step = 16.0 / (g - 1)
    s = (0.5 ** 0.5) / step
    offs = (jnp.linspace(0.0, 16.0, g) * s).reshape(1, g).astype(jnp.float32)
    s_arr = jnp.full((1, 1), s, jnp.float32)

    ones4 = jnp.ones((4, g), jnp.float32)
    packed = _edge_mlp(
        gp, edge_attr,
        msg_W0[2 * d:2 * d + g], msg_W0[2 * d + g:], msg_b0.reshape(1, -1),
        msg_W1, msg_b1.reshape(1, -1), msg_W2, msg_b2.reshape(1, -1),
        coord_W.reshape(d, 1), coord_b.reshape(1, 1), offs, s_arr,
        ones4, node_W0[d:], eb=4000)

    pm = _sc_scatter(packed, dst, n)

    x_update, pos_update = _node_update(
        x, pm, node_W0[:d], node_b0.reshape(1, -1),
        node_W1, node_b1.reshape(1, -1), node_W2, node_b2.reshape(1, -1),
        out_W, out_b.reshape(1, -1), nb=2000)
    return (x_update, pos_update)


# speedup vs baseline: 7.1470x; 1.2148x over previous
import functools

import jax
import jax.numpy as jnp
from jax import lax
from jax.experimental import pallas as pl
from jax.experimental.pallas import tpu as pltpu
from jax.experimental.pallas import tpu_sc as plsc

_NC = 2
_NS = 16


def _lrelu(v):
    return jnp.where(v >= 0, v, 0.2 * v)


def _proj_body(x_ref, p4_ref, wd_ref, ws_ref, t_ref):
    x = x_ref[...]
    p4 = p4_ref[...]
    nb = x.shape[0]
    t_ref[...] = jnp.concatenate(
        [jnp.dot(x, wd_ref[...], preferred_element_type=jnp.float32),
         jnp.dot(x, ws_ref[...], preferred_element_type=jnp.float32),
         p4, jnp.zeros((nb, 60), jnp.float32)], axis=1)


def _node_table(x, pos4, wd, ws, nb):
    n, d = x.shape
    h = wd.shape[1]
    return pl.pallas_call(
        _proj_body,
        grid=(n // nb,),
        in_specs=[
            pl.BlockSpec((nb, d), lambda i: (i, 0)),
            pl.BlockSpec((nb, 4), lambda i: (i, 0)),
            pl.BlockSpec((d, h), lambda i: (0, 0)),
            pl.BlockSpec((d, h), lambda i: (0, 0)),
        ],
        out_specs=pl.BlockSpec((nb, 128), lambda i: (i, 0)),
        out_shape=jax.ShapeDtypeStruct((n, 128), jnp.float32),
    )(x, pos4, wd, ws)


def _sc_gather(tbl, src, dst):
    e = src.shape[0]
    k = 80
    per_w = e // (_NC * _NS)
    chunks = per_w // k
    mesh = plsc.VectorSubcoreMesh(
        core_axis_name="c", subcore_axis_name="s",
        num_cores=_NC, num_subcores=_NS)

    nbuf = 3
    nob = 2

    @functools.partial(
        pl.kernel,
        out_type=jax.ShapeDtypeStruct((e, 128), jnp.float32),
        mesh=mesh,
        scratch_types=[
            pltpu.VMEM((nbuf, k), jnp.int32),
            pltpu.VMEM((nbuf, k), jnp.int32),
            pltpu.VMEM((nbuf, k, 128), jnp.float32),
            pltpu.VMEM((nbuf, k, 128), jnp.float32),
            pltpu.VMEM((nob, k, 128), jnp.float32),
            pltpu.SemaphoreType.DMA((nbuf,)),
            pltpu.SemaphoreType.DMA((nbuf,)),
            pltpu.SemaphoreType.DMA((nob,)),
        ],
    )
    def kern(tbl_hbm, src_hbm, dst_hbm, out_hbm,
             isrc, idst, bufd, bufs, bufo, isem, gsem, osem):
        wid = lax.axis_index("s") * _NC + lax.axis_index("c")
        base = wid * per_w
        zero = jnp.zeros((16,), jnp.float32)

        def zrow(r, carry):
            for b in range(nob):
                for cc in range(5):
                    bufo[b, r, pl.ds(48 + cc * 16, 16)] = zero
            return carry

        lax.fori_loop(0, k, zrow, 0)

        def fetch_idx(j):
            p = lax.rem(j, nbuf)
            off = base + j * k
            pltpu.async_copy(dst_hbm.at[pl.ds(off, k)], idst.at[p],
                             isem.at[p])
            pltpu.async_copy(src_hbm.at[pl.ds(off, k)], isrc.at[p],
                             isem.at[p])

        def wait_idx(j):
            p = lax.rem(j, nbuf)
            pltpu.make_async_copy(dst_hbm.at[pl.ds(0, k)], idst.at[p],
                                  isem.at[p]).wait()
            pltpu.make_async_copy(src_hbm.at[pl.ds(0, k)], isrc.at[p],
                                  isem.at[p]).wait()

        def gathers(j):
            p = lax.rem(j, nbuf)
            pltpu.async_copy(tbl_hbm.at[idst.at[p]], bufd.at[p], gsem.at[p])
            pltpu.async_copy(tbl_hbm.at[isrc.at[p]], bufs.at[p], gsem.at[p])

        def wait_gathers(j):
            p = lax.rem(j, nbuf)
            pltpu.make_async_copy(tbl_hbm.at[idst.at[p]], bufd.at[p],
                                  gsem.at[p]).wait()
            pltpu.make_async_copy(tbl_hbm.at[isrc.at[p]], bufs.at[p],
                                  gsem.at[p]).wait()

        def wait_wo(j):
            po = lax.rem(j, nob)
            pltpu.make_async_copy(bufo.at[po], out_hbm.at[pl.ds(0, k)],
                                  osem.at[po]).wait()

        fetch_idx(0)
        fetch_idx(1)
        wait_idx(0)
        gathers(0)

        def body(j, carry):
            p = lax.rem(j, nbuf)
            po = lax.rem(j, nob)

            @pl.when(j + 2 < chunks)
            def _():
                fetch_idx(j + 2)

            @pl.when(j + 1 < chunks)
            def _():
                wait_idx(j + 1)
                gathers(j + 1)

            wait_gathers(j)

            @pl.when(j >= nob)
            def _():
                wait_wo(j - nob)

            def row(r, c2):
                bufo[po, r, pl.ds(0, 16)] = (bufd[p, r, pl.ds(0, 16)]
                                             + bufs[p, r, pl.ds(32, 16)])
                bufo[po, r, pl.ds(16, 16)] = (bufd[p, r, pl.ds(16, 16)]
                                              + bufs[p, r, pl.ds(48, 16)])
                bufo[po, r, pl.ds(32, 16)] = (bufd[p, r, pl.ds(64, 16)]
                                              - bufs[p, r, pl.ds(64, 16)])
                return c2

            lax.fori_loop(0, k, row, 0)
            off = base + j * k
            pltpu.async_copy(bufo.at[po], out_hbm.at[pl.ds(off, k)],
                             osem.at[po])
            return carry

        lax.fori_loop(0, chunks, body, 0)
        wait_wo(chunks - 2)
        wait_wo(chunks - 1)

    return kern(tbl, src, dst)


def _edge_body(gp_ref, ea_ref, wsm_ref, wea_ref, b0_ref, w1_ref,
               b1_ref, w2_ref, b2_ref, cw_ref, cb_ref, offs_ref, s_ref,
               ones4_ref, w0b_ref, out_ref):
    gp = gp_ref[...]
    d4 = gp[:, 32:36]
    dd32 = jnp.dot(d4 * d4, ones4_ref[...],
                   preferred_element_type=jnp.float32)
    dist32 = jnp.sqrt(dd32)
    s = s_ref[0, 0]
    t = dist32 * s - offs_ref[...]
    sm = jnp.exp(-(t * t))
    pre0 = (gp[:, :32]
            + jnp.dot(sm, wsm_ref[...], preferred_element_type=jnp.float32)
            + jnp.dot(ea_ref[...], wea_ref[...], preferred_element_type=jnp.float32)
            + b0_ref[...])
    h1 = _lrelu(pre0)
    h2 = _lrelu(jnp.dot(h1, w1_ref[...], preferred_element_type=jnp.float32)
                + b1_ref[...])
    z = jnp.dot(h2, w2_ref[...], preferred_element_type=jnp.float32) + b2_ref[...]
    msg = _lrelu(z)
    m32 = jnp.dot(msg, w0b_ref[...], preferred_element_type=jnp.float32)
    ew = jnp.dot(msg, cw_ref[...], preferred_element_type=jnp.float32) \
        + cb_ref[0, 0]
    q = ew / (dist32[:, 0:1] + 1e-6)
    eb = msg.shape[0]
    w4 = jnp.concatenate(
        [d4[:, :3] * q, jnp.ones((eb, 1), jnp.float32)], axis=1)
    out_ref[...] = jnp.concatenate(
        [m32, w4, jnp.zeros((eb, 92), jnp.float32)], axis=1)


def _edge_mlp(gp, ea, wsm, wea, b0, w1, b1, w2, b2, cw, cb, offs, s,
              ones4, w0b, eb):
    e = gp.shape[0]
    de = ea.shape[1]
    g = offs.shape[1]
    dout = w2.shape[1]
    const = lambda i: (0, 0)
    return pl.pallas_call(
        _edge_body,
        grid=(e // eb,),
        in_specs=[
            pl.BlockSpec((eb, 128), lambda i: (i, 0)),
            pl.BlockSpec((eb, de), lambda i: (i, 0)),
            pl.BlockSpec((g, 32), const),
            pl.BlockSpec((de, 32), const),
            pl.BlockSpec((1, 32), const),
            pl.BlockSpec((32, 32), const),
            pl.BlockSpec((1, 32), const),
            pl.BlockSpec((32, dout), const),
            pl.BlockSpec((1, dout), const),
            pl.BlockSpec((dout, 1), const),
            pl.BlockSpec((1, 1), const),
            pl.BlockSpec((1, g), const),
            pl.BlockSpec((1, 1), const),
            pl.BlockSpec((4, g), const),
            pl.BlockSpec((dout, 32), const),
        ],
        out_specs=pl.BlockSpec((eb, 128), lambda i: (i, 0)),
        out_shape=jax.ShapeDtypeStruct((e, 128), jnp.float32),
    )(gp, ea, wsm, wea, b0, w1, b1, w2, b2, cw, cb, offs, s, ones4, w0b)


def _sc_scatter(rows_in, dst, n):
    e = rows_in.shape[0]
    k = 80
    per_w = e // (_NC * _NS)
    chunks = per_w // k
    zrows = 80
    nblk = n // zrows
    mesh = plsc.VectorSubcoreMesh(
        core_axis_name="c", subcore_axis_name="s",
        num_cores=_NC, num_subcores=_NS)

    nbuf = 3

    @functools.partial(
        pl.kernel,
        out_type=jax.ShapeDtypeStruct((_NC, n, 128), jnp.float32),
        mesh=mesh,
        scratch_types=[
            pltpu.VMEM((nbuf, k), jnp.int32),
            pltpu.VMEM((nbuf, k, 128), jnp.float32),
            pltpu.VMEM((zrows, 128), jnp.float32),
            pltpu.VMEM_SHARED((n, 128), jnp.float32),
            pltpu.SemaphoreType.DMA((nbuf,)),
            pltpu.SemaphoreType.DMA((nbuf,)),
        ],
    )
    def kern(rows_hbm, dst_hbm, out_hbm, idx, rows, zbuf, acc, fsem, ssem):
        c = lax.axis_index("c")
        s = lax.axis_index("s")
        wid = s * _NC + c
        zero = jnp.zeros((16,), jnp.float32)

        def zr(r, carry):
            for cc in range(8):
                zbuf[r, pl.ds(cc * 16, 16)] = zero
            return carry

        lax.fori_loop(0, zrows, zr, 0)

        def zc(i, carry):
            blk = i * _NS + s

            @pl.when(blk < nblk)
            def _():
                pltpu.sync_copy(zbuf, acc.at[pl.ds(blk * zrows, zrows)])

            return carry

        lax.fori_loop(0, (nblk + _NS - 1) // _NS, zc, 0)
        plsc.subcore_barrier()

        base = wid * per_w

        def fetch(j):
            p = lax.rem(j, nbuf)
            off = base + j * k
            pltpu.async_copy(dst_hbm.at[pl.ds(off, k)], idx.at[p], fsem.at[p])
            pltpu.async_copy(rows_hbm.at[pl.ds(off, k)], rows.at[p],
                             fsem.at[p])

        def wait_fetch(j):
            p = lax.rem(j, nbuf)
            pltpu.make_async_copy(dst_hbm.at[pl.ds(0, k)], idx.at[p],
                                  fsem.at[p]).wait()
            pltpu.make_async_copy(rows_hbm.at[pl.ds(0, k)], rows.at[p],
                                  fsem.at[p]).wait()

        def scat(j):
            p = lax.rem(j, nbuf)
            pltpu.async_copy(rows.at[p], acc.at[idx.at[p]], ssem.at[p],
                             add=True)

        def wait_scat(j):
            p = lax.rem(j, nbuf)
            pltpu.make_async_copy(rows.at[p], acc.at[idx.at[p]],
                                  ssem.at[p]).wait()

        fetch(0)
        fetch(1)

        def body(j, carry):
            @pl.when(j >= 1)
            def _():
                wait_scat(j - 1)

            @pl.when(j + 2 < chunks)
            def _():
                fetch(j + 2)

            wait_fetch(j)
            scat(j)
            return carry

        lax.fori_loop(0, chunks, body, 0)
        wait_scat(chunks - 1)
        plsc.subcore_barrier()

        def wb(i, carry):
            blk = i * _NS + s

            @pl.when(blk < nblk)
            def _():
                pltpu.sync_copy(acc.at[pl.ds(blk * zrows, zrows)],
                                out_hbm.at[c].at[pl.ds(blk * zrows, zrows)])

            return carry

        lax.fori_loop(0, (nblk + _NS - 1) // _NS, wb, 0)

    return kern(rows_in, dst)


def _final_body(x_ref, pm_ref, w0a_ref, b0_ref, w1_ref,
                b1_ref, w2_ref, b2_ref, wo_ref, bo_ref, xu_ref, pu_ref):
    agg = pm_ref[0] + pm_ref[1]
    m32 = agg[:, 0:32]
    num = agg[:, 32:35]
    cnt = agg[:, 35:36]
    pu_ref[...] = num / jnp.maximum(cnt, 1.0)
    h = _lrelu(jnp.dot(x_ref[...], w0a_ref[...], preferred_element_type=jnp.float32)
               + m32 + b0_ref[...])
    h = _lrelu(jnp.dot(h, w1_ref[...], preferred_element_type=jnp.float32)
               + b1_ref[...])
    h = _lrelu(jnp.dot(h, w2_ref[...], preferred_element_type=jnp.float32)
               + b2_ref[...])
    xu_ref[...] = (jnp.dot(h, wo_ref[...], preferred_element_type=jnp.float32)
                   + bo_ref[...])


def _node_update(x, pm, w0a, b0, w1, b1, w2, b2, wo, bo, nb):
    n, d = x.shape
    hh = w0a.shape[1]
    const = lambda i: (0, 0)
    return pl.pallas_call(
        _final_body,
        grid=(n // nb,),
        in_specs=[
            pl.BlockSpec((nb, d), lambda i: (i, 0)),
            pl.BlockSpec((2, nb, 128), lambda i: (0, i, 0)),
            pl.BlockSpec((d, hh), const),
            pl.BlockSpec((1, hh), const),
            pl.BlockSpec((hh, hh), const),
            pl.BlockSpec((1, hh), const),
            pl.BlockSpec((hh, d), const),
            pl.BlockSpec((1, d), const),
            pl.BlockSpec((d, d), const),
            pl.BlockSpec((1, d), const),
        ],
        out_specs=[
            pl.BlockSpec((nb, d), lambda i: (i, 0)),
            pl.BlockSpec((nb, 3), lambda i: (i, 0)),
        ],
        out_shape=[
            jax.ShapeDtypeStruct((n, d), jnp.float32),
            jax.ShapeDtypeStruct((n, 3), jnp.float32),
        ],
    )(x, pm, w0a, b0, w1, b1, w2, b2, wo, bo)


def kernel(x, pos, edge_index, batch, edge_attr, msg_W0, msg_b0, msg_W1,
           msg_b1, msg_W2, msg_b2, coord_W, coord_b, node_W0, node_b0,
           node_W1, node_b1, node_W2, node_b2, out_W, out_b):
    n, d = x.shape
    g = msg_W0.shape[0] - 2 * d - edge_attr.shape[1]
    src = edge_index[0]
    dst = edge_index[1]

    pos4 = jnp.pad(pos, ((0, 0), (0, 1)))
    tbl = _node_table(x, pos4, msg_W0[:d], msg_W0[d:2 * d], nb=2000)

    gp = _sc_gather(tbl, src, dst)

    step = 16.0 / (g - 1)
    s = (0.5 ** 0.5) / step
    offs = (jnp.linspace(0.0, 16.0, g) * s).reshape(1, g).astype(jnp.float32)
    s_arr = jnp.full((1, 1), s, jnp.float32)

    ones4 = jnp.ones((4, g), jnp.float32)
    packed = _edge_mlp(
        gp, edge_attr,
        msg_W0[2 * d:2 * d + g], msg_W0[2 * d + g:], msg_b0.reshape(1, -1),
        msg_W1, msg_b1.reshape(1, -1), msg_W2, msg_b2.reshape(1, -1),
        coord_W.reshape(d, 1), coord_b.reshape(1, 1), offs, s_arr,
        ones4, node_W0[d:], eb=4000)

    pm = _sc_scatter(packed, dst, n)

    x_update, pos_update = _node_update(
        x, pm, node_W0[:d], node_b0.reshape(1, -1),
        node_W1, node_b1.reshape(1, -1), node_W2, node_b2.reshape(1, -1),
        out_W, out_b.reshape(1, -1), nb=2000)
    return (x_update, pos_update)


# speedup vs baseline: 8.2194x; 1.1500x over previous
import functools

import jax
import jax.numpy as jnp
from jax import lax
from jax.experimental import pallas as pl
from jax.experimental.pallas import tpu as pltpu
from jax.experimental.pallas import tpu_sc as plsc

_NC = 2
_NS = 16


def _lrelu(v):
    return jnp.where(v >= 0, v, 0.2 * v)


def _proj_body(x_ref, p4_ref, wd_ref, ws_ref, t_ref):
    x = x_ref[...]
    p4 = p4_ref[...]
    nb = x.shape[0]
    t_ref[...] = jnp.concatenate(
        [jnp.dot(x, wd_ref[...], preferred_element_type=jnp.float32),
         jnp.dot(x, ws_ref[...], preferred_element_type=jnp.float32),
         p4, jnp.zeros((nb, 60), jnp.float32)], axis=1)


def _node_table(x, pos4, wd, ws, nb):
    n, d = x.shape
    h = wd.shape[1]
    return pl.pallas_call(
        _proj_body,
        grid=(n // nb,),
        in_specs=[
            pl.BlockSpec((nb, d), lambda i: (i, 0)),
            pl.BlockSpec((nb, 4), lambda i: (i, 0)),
            pl.BlockSpec((d, h), lambda i: (0, 0)),
            pl.BlockSpec((d, h), lambda i: (0, 0)),
        ],
        out_specs=pl.BlockSpec((nb, 128), lambda i: (i, 0)),
        out_shape=jax.ShapeDtypeStruct((n, 128), jnp.float32),
    )(x, pos4, wd, ws)


def _sc_gather(tbl, src, dst):
    e = src.shape[0]
    k = 80
    per_w = e // (_NC * _NS)
    chunks = per_w // k
    mesh = plsc.VectorSubcoreMesh(
        core_axis_name="c", subcore_axis_name="s",
        num_cores=_NC, num_subcores=_NS)

    nbuf = 3
    nob = 2

    @functools.partial(
        pl.kernel,
        out_type=jax.ShapeDtypeStruct((e, 128), jnp.float32),
        mesh=mesh,
        scratch_types=[
            pltpu.VMEM((nbuf, k), jnp.int32),
            pltpu.VMEM((nbuf, k), jnp.int32),
            pltpu.VMEM((nbuf, k, 128), jnp.float32),
            pltpu.VMEM((nbuf, k, 128), jnp.float32),
            pltpu.VMEM((nob, k, 128), jnp.float32),
            pltpu.SemaphoreType.DMA((nbuf,)),
            pltpu.SemaphoreType.DMA((nbuf,)),
            pltpu.SemaphoreType.DMA((nob,)),
        ],
    )
    def kern(tbl_hbm, src_hbm, dst_hbm, out_hbm,
             isrc, idst, bufd, bufs, bufo, isem, gsem, osem):
        wid = lax.axis_index("s") * _NC + lax.axis_index("c")
        base = wid * per_w
        zero = jnp.zeros((16,), jnp.float32)

        def zrow(r, carry):
            for b in range(nob):
                for cc in range(5):
                    bufo[b, r, pl.ds(48 + cc * 16, 16)] = zero
            return carry

        lax.fori_loop(0, k, zrow, 0)

        def fetch_idx(j):
            p = lax.rem(j, nbuf)
            off = base + j * k
            pltpu.async_copy(dst_hbm.at[pl.ds(off, k)], idst.at[p],
                             isem.at[p])
            pltpu.async_copy(src_hbm.at[pl.ds(off, k)], isrc.at[p],
                             isem.at[p])

        def wait_idx(j):
            p = lax.rem(j, nbuf)
            pltpu.make_async_copy(dst_hbm.at[pl.ds(0, k)], idst.at[p],
                                  isem.at[p]).wait()
            pltpu.make_async_copy(src_hbm.at[pl.ds(0, k)], isrc.at[p],
                                  isem.at[p]).wait()

        def gathers(j):
            p = lax.rem(j, nbuf)
            pltpu.async_copy(tbl_hbm.at[idst.at[p]], bufd.at[p], gsem.at[p])
            pltpu.async_copy(tbl_hbm.at[isrc.at[p]], bufs.at[p], gsem.at[p])

        def wait_gathers(j):
            p = lax.rem(j, nbuf)
            pltpu.make_async_copy(tbl_hbm.at[idst.at[p]], bufd.at[p],
                                  gsem.at[p]).wait()
            pltpu.make_async_copy(tbl_hbm.at[isrc.at[p]], bufs.at[p],
                                  gsem.at[p]).wait()

        def wait_wo(j):
            po = lax.rem(j, nob)
            pltpu.make_async_copy(bufo.at[po], out_hbm.at[pl.ds(0, k)],
                                  osem.at[po]).wait()

        fetch_idx(0)
        fetch_idx(1)
        wait_idx(0)
        gathers(0)

        def body(j, carry):
            p = lax.rem(j, nbuf)
            po = lax.rem(j, nob)

            @pl.when(j + 2 < chunks)
            def _():
                fetch_idx(j + 2)

            @pl.when(j + 1 < chunks)
            def _():
                wait_idx(j + 1)
                gathers(j + 1)

            wait_gathers(j)

            @pl.when(j >= nob)
            def _():
                wait_wo(j - nob)

            def row(rr, c2):
                for u in range(4):
                    r = rr * 4 + u
                    bufo[po, r, pl.ds(0, 16)] = (bufd[p, r, pl.ds(0, 16)]
                                                 + bufs[p, r, pl.ds(32, 16)])
                    bufo[po, r, pl.ds(16, 16)] = (bufd[p, r, pl.ds(16, 16)]
                                                  + bufs[p, r, pl.ds(48, 16)])
                    bufo[po, r, pl.ds(32, 16)] = (bufd[p, r, pl.ds(64, 16)]
                                                  - bufs[p, r, pl.ds(64, 16)])
                return c2

            lax.fori_loop(0, k // 4, row, 0)
            off = base + j * k
            pltpu.async_copy(bufo.at[po], out_hbm.at[pl.ds(off, k)],
                             osem.at[po])
            return carry

        lax.fori_loop(0, chunks, body, 0)
        wait_wo(chunks - 2)
        wait_wo(chunks - 1)

    return kern(tbl, src, dst)


def _edge_body(gp_ref, ea_ref, wsm_ref, wea_ref, b0_ref, w1_ref,
               b1_ref, w2_ref, b2_ref, cw_ref, cb_ref, offs_ref, s_ref,
               ones4_ref, w0b_ref, oh4_ref, out_ref):
    gp = gp_ref[...]
    d4 = gp[:, 32:36]
    dd32 = jnp.dot(d4 * d4, ones4_ref[...],
                   preferred_element_type=jnp.float32)
    dist32 = dd32 * lax.rsqrt(dd32 + 1e-30)
    s = s_ref[0, 0]
    t = dist32 * s - offs_ref[...]
    sm = jnp.exp(-(t * t))
    pre0 = (gp[:, :32]
            + jnp.dot(sm, wsm_ref[...], preferred_element_type=jnp.float32)
            + jnp.dot(ea_ref[...], wea_ref[...], preferred_element_type=jnp.float32)
            + b0_ref[...])
    h1 = _lrelu(pre0)
    h2 = _lrelu(jnp.dot(h1, w1_ref[...], preferred_element_type=jnp.float32)
                + b1_ref[...])
    z = jnp.dot(h2, w2_ref[...], preferred_element_type=jnp.float32) + b2_ref[...]
    msg = _lrelu(z)
    m32 = jnp.dot(msg, w0b_ref[...], preferred_element_type=jnp.float32)
    ew = jnp.dot(msg, cw_ref[...], preferred_element_type=jnp.float32) \
        + cb_ref[0, 0]
    q = ew / (dist32[:, 0:1] + 1e-6)
    out_ref[:, 0:32] = m32
    out_ref[:, 32:36] = d4 * q + oh4_ref[...]


def _edge_mlp(gp, ea, wsm, wea, b0, w1, b1, w2, b2, cw, cb, offs, s,
              ones4, w0b, oh4, eb):
    e = gp.shape[0]
    de = ea.shape[1]
    g = offs.shape[1]
    dout = w2.shape[1]
    const = lambda i: (0, 0)
    return pl.pallas_call(
        _edge_body,
        grid=(e // eb,),
        in_specs=[
            pl.BlockSpec((eb, 128), lambda i: (i, 0)),
            pl.BlockSpec((eb, de), lambda i: (i, 0)),
            pl.BlockSpec((g, 32), const),
            pl.BlockSpec((de, 32), const),
            pl.BlockSpec((1, 32), const),
            pl.BlockSpec((32, 32), const),
            pl.BlockSpec((1, 32), const),
            pl.BlockSpec((32, dout), const),
            pl.BlockSpec((1, dout), const),
            pl.BlockSpec((dout, 1), const),
            pl.BlockSpec((1, 1), const),
            pl.BlockSpec((1, g), const),
            pl.BlockSpec((1, 1), const),
            pl.BlockSpec((4, g), const),
            pl.BlockSpec((dout, 32), const),
            pl.BlockSpec((1, 4), const),
        ],
        out_specs=pl.BlockSpec((eb, 128), lambda i: (i, 0)),
        out_shape=jax.ShapeDtypeStruct((e, 128), jnp.float32),
    )(gp, ea, wsm, wea, b0, w1, b1, w2, b2, cw, cb, offs, s, ones4, w0b, oh4)


def _sc_scatter(rows_in, dst, n):
    e = rows_in.shape[0]
    k = 80
    per_w = e // (_NC * _NS)
    chunks = per_w // k
    zrows = 80
    nblk = n // zrows
    mesh = plsc.VectorSubcoreMesh(
        core_axis_name="c", subcore_axis_name="s",
        num_cores=_NC, num_subcores=_NS)

    nbuf = 3

    @functools.partial(
        pl.kernel,
        out_type=jax.ShapeDtypeStruct((_NC, n, 128), jnp.float32),
        mesh=mesh,
        scratch_types=[
            pltpu.VMEM((nbuf, k), jnp.int32),
            pltpu.VMEM((nbuf, k, 128), jnp.float32),
            pltpu.VMEM((zrows, 128), jnp.float32),
            pltpu.VMEM_SHARED((n, 128), jnp.float32),
            pltpu.SemaphoreType.DMA((nbuf,)),
            pltpu.SemaphoreType.DMA((nbuf,)),
        ],
    )
    def kern(rows_hbm, dst_hbm, out_hbm, idx, rows, zbuf, acc, fsem, ssem):
        c = lax.axis_index("c")
        s = lax.axis_index("s")
        wid = s * _NC + c
        zero = jnp.zeros((16,), jnp.float32)

        def zr(r, carry):
            for cc in range(8):
                zbuf[r, pl.ds(cc * 16, 16)] = zero
            return carry

        lax.fori_loop(0, zrows, zr, 0)

        def zc(i, carry):
            blk = i * _NS + s

            @pl.when(blk < nblk)
            def _():
                pltpu.sync_copy(zbuf, acc.at[pl.ds(blk * zrows, zrows)])

            return carry

        lax.fori_loop(0, (nblk + _NS - 1) // _NS, zc, 0)
        plsc.subcore_barrier()

        base = wid * per_w

        def fetch(j):
            p = lax.rem(j, nbuf)
            off = base + j * k
            pltpu.async_copy(dst_hbm.at[pl.ds(off, k)], idx.at[p], fsem.at[p])
            pltpu.async_copy(rows_hbm.at[pl.ds(off, k)], rows.at[p],
                             fsem.at[p])

        def wait_fetch(j):
            p = lax.rem(j, nbuf)
            pltpu.make_async_copy(dst_hbm.at[pl.ds(0, k)], idx.at[p],
                                  fsem.at[p]).wait()
            pltpu.make_async_copy(rows_hbm.at[pl.ds(0, k)], rows.at[p],
                                  fsem.at[p]).wait()

        def scat(j):
            p = lax.rem(j, nbuf)
            pltpu.async_copy(rows.at[p], acc.at[idx.at[p]], ssem.at[p],
                             add=True)

        def wait_scat(j):
            p = lax.rem(j, nbuf)
            pltpu.make_async_copy(rows.at[p], acc.at[idx.at[p]],
                                  ssem.at[p]).wait()

        fetch(0)
        fetch(1)

        def body(j, carry):
            @pl.when(j >= 1)
            def _():
                wait_scat(j - 1)

            @pl.when(j + 2 < chunks)
            def _():
                fetch(j + 2)

            wait_fetch(j)
            scat(j)
            return carry

        lax.fori_loop(0, chunks, body, 0)
        wait_scat(chunks - 1)
        plsc.subcore_barrier()

        def wb(i, carry):
            blk = i * _NS + s

            @pl.when(blk < nblk)
            def _():
                pltpu.sync_copy(acc.at[pl.ds(blk * zrows, zrows)],
                                out_hbm.at[c].at[pl.ds(blk * zrows, zrows)])

            return carry

        lax.fori_loop(0, (nblk + _NS - 1) // _NS, wb, 0)

    return kern(rows_in, dst)


def _final_body(x_ref, pm_ref, w0a_ref, b0_ref, w1_ref,
                b1_ref, w2_ref, b2_ref, wo_ref, bo_ref, xu_ref, pu_ref):
    agg = pm_ref[0] + pm_ref[1]
    m32 = agg[:, 0:32]
    num = agg[:, 32:35]
    cnt = agg[:, 35:36]
    pu_ref[...] = num / jnp.maximum(cnt, 1.0)
    h = _lrelu(jnp.dot(x_ref[...], w0a_ref[...], preferred_element_type=jnp.float32)
               + m32 + b0_ref[...])
    h = _lrelu(jnp.dot(h, w1_ref[...], preferred_element_type=jnp.float32)
               + b1_ref[...])
    h = _lrelu(jnp.dot(h, w2_ref[...], preferred_element_type=jnp.float32)
               + b2_ref[...])
    xu_ref[...] = (jnp.dot(h, wo_ref[...], preferred_element_type=jnp.float32)
                   + bo_ref[...])


def _node_update(x, pm, w0a, b0, w1, b1, w2, b2, wo, bo, nb):
    n, d = x.shape
    hh = w0a.shape[1]
    const = lambda i: (0, 0)
    return pl.pallas_call(
        _final_body,
        grid=(n // nb,),
        in_specs=[
            pl.BlockSpec((nb, d), lambda i: (i, 0)),
            pl.BlockSpec((2, nb, 128), lambda i: (0, i, 0)),
            pl.BlockSpec((d, hh), const),
            pl.BlockSpec((1, hh), const),
            pl.BlockSpec((hh, hh), const),
            pl.BlockSpec((1, hh), const),
            pl.BlockSpec((hh, d), const),
            pl.BlockSpec((1, d), const),
            pl.BlockSpec((d, d), const),
            pl.BlockSpec((1, d), const),
        ],
        out_specs=[
            pl.BlockSpec((nb, d), lambda i: (i, 0)),
            pl.BlockSpec((nb, 3), lambda i: (i, 0)),
        ],
        out_shape=[
            jax.ShapeDtypeStruct((n, d), jnp.float32),
            jax.ShapeDtypeStruct((n, 3), jnp.float32),
        ],
    )(x, pm, w0a, b0, w1, b1, w2, b2, wo, bo)


def kernel(x, pos, edge_index, batch, edge_attr, msg_W0, msg_b0, msg_W1,
           msg_b1, msg_W2, msg_b2, coord_W, coord_b, node_W0, node_b0,
           node_W1, node_b1, node_W2, node_b2, out_W, out_b):
    n, d = x.shape
    g = msg_W0.shape[0] - 2 * d - edge_attr.shape[1]
    src = edge_index[0]
    dst = edge_index[1]

    pos4 = jnp.pad(pos, ((0, 0), (0, 1)))
    tbl = _node_table(x, pos4, msg_W0[:d], msg_W0[d:2 * d], nb=2000)

    gp = _sc_gather(tbl, src, dst)

    step = 16.0 / (g - 1)
    s = (0.5 ** 0.5) / step
    offs = (jnp.linspace(0.0, 16.0, g) * s).reshape(1, g).astype(jnp.float32)
    s_arr = jnp.full((1, 1), s, jnp.float32)

    ones4 = jnp.ones((4, g), jnp.float32)
    oh4 = jnp.array([[0.0, 0.0, 0.0, 1.0]], jnp.float32)
    packed = _edge_mlp(
        gp, edge_attr,
        msg_W0[2 * d:2 * d + g], msg_W0[2 * d + g:], msg_b0.reshape(1, -1),
        msg_W1, msg_b1.reshape(1, -1), msg_W2, msg_b2.reshape(1, -1),
        coord_W.reshape(d, 1), coord_b.reshape(1, 1), offs, s_arr,
        ones4, node_W0[d:], oh4, eb=8000)

    pm = _sc_scatter(packed, dst, n)

    x_update, pos_update = _node_update(
        x, pm, node_W0[:d], node_b0.reshape(1, -1),
        node_W1, node_b1.reshape(1, -1), node_W2, node_b2.reshape(1, -1),
        out_W, out_b.reshape(1, -1), nb=2000)
    return (x_update, pos_update)


# speedup vs baseline: 8.5316x; 1.0380x over previous
import functools

import jax
import jax.numpy as jnp
from jax import lax
from jax.experimental import pallas as pl
from jax.experimental.pallas import tpu as pltpu
from jax.experimental.pallas import tpu_sc as plsc

_NC = 2
_NS = 16


def _lrelu(v):
    return jnp.where(v >= 0, v, 0.2 * v)


def _proj_body(x_ref, p4_ref, wd_ref, ws_ref, t_ref):
    x = x_ref[...]
    p4 = p4_ref[...]
    nb = x.shape[0]
    t_ref[...] = jnp.concatenate(
        [jnp.dot(x, wd_ref[...], preferred_element_type=jnp.float32),
         jnp.dot(x, ws_ref[...], preferred_element_type=jnp.float32),
         p4, jnp.zeros((nb, 60), jnp.float32)], axis=1)


def _node_table(x, pos4, wd, ws, nb):
    n, d = x.shape
    h = wd.shape[1]
    return pl.pallas_call(
        _proj_body,
        grid=(n // nb,),
        in_specs=[
            pl.BlockSpec((nb, d), lambda i: (i, 0)),
            pl.BlockSpec((nb, 4), lambda i: (i, 0)),
            pl.BlockSpec((d, h), lambda i: (0, 0)),
            pl.BlockSpec((d, h), lambda i: (0, 0)),
        ],
        out_specs=pl.BlockSpec((nb, 128), lambda i: (i, 0)),
        out_shape=jax.ShapeDtypeStruct((n, 128), jnp.float32),
    )(x, pos4, wd, ws)


def _sc_gather(tbl, src, dst, lo, hi):
    e = hi - lo
    k = 40
    per_w = e // (_NC * _NS)
    chunks = per_w // k
    mesh = plsc.VectorSubcoreMesh(
        core_axis_name="c", subcore_axis_name="s",
        num_cores=_NC, num_subcores=_NS)

    nbuf = 3
    nob = 2

    @functools.partial(
        pl.kernel,
        out_type=jax.ShapeDtypeStruct((e, 128), jnp.float32),
        mesh=mesh,
        scratch_types=[
            pltpu.VMEM((nbuf, k), jnp.int32),
            pltpu.VMEM((nbuf, k), jnp.int32),
            pltpu.VMEM((nbuf, k, 128), jnp.float32),
            pltpu.VMEM((nbuf, k, 128), jnp.float32),
            pltpu.VMEM((nob, k, 128), jnp.float32),
            pltpu.SemaphoreType.DMA((nbuf,)),
            pltpu.SemaphoreType.DMA((nbuf,)),
            pltpu.SemaphoreType.DMA((nob,)),
        ],
    )
    def kern(tbl_hbm, src_hbm, dst_hbm, out_hbm,
             isrc, idst, bufd, bufs, bufo, isem, gsem, osem):
        wid = lax.axis_index("s") * _NC + lax.axis_index("c")
        base = lo + wid * per_w
        zero = jnp.zeros((16,), jnp.float32)

        def zrow(r, carry):
            for b in range(nob):
                for cc in range(5):
                    bufo[b, r, pl.ds(48 + cc * 16, 16)] = zero
            return carry

        lax.fori_loop(0, k, zrow, 0)

        def fetch_idx(j):
            p = lax.rem(j, nbuf)
            off = base + j * k
            pltpu.async_copy(dst_hbm.at[pl.ds(off, k)], idst.at[p],
                             isem.at[p])
            pltpu.async_copy(src_hbm.at[pl.ds(off, k)], isrc.at[p],
                             isem.at[p])

        def wait_idx(j):
            p = lax.rem(j, nbuf)
            pltpu.make_async_copy(dst_hbm.at[pl.ds(0, k)], idst.at[p],
                                  isem.at[p]).wait()
            pltpu.make_async_copy(src_hbm.at[pl.ds(0, k)], isrc.at[p],
                                  isem.at[p]).wait()

        def gathers(j):
            p = lax.rem(j, nbuf)
            pltpu.async_copy(tbl_hbm.at[idst.at[p]], bufd.at[p], gsem.at[p])
            pltpu.async_copy(tbl_hbm.at[isrc.at[p]], bufs.at[p], gsem.at[p])

        def wait_gathers(j):
            p = lax.rem(j, nbuf)
            pltpu.make_async_copy(tbl_hbm.at[idst.at[p]], bufd.at[p],
                                  gsem.at[p]).wait()
            pltpu.make_async_copy(tbl_hbm.at[isrc.at[p]], bufs.at[p],
                                  gsem.at[p]).wait()

        def wait_wo(j):
            po = lax.rem(j, nob)
            pltpu.make_async_copy(bufo.at[po], out_hbm.at[pl.ds(0, k)],
                                  osem.at[po]).wait()

        fetch_idx(0)
        fetch_idx(1)
        wait_idx(0)
        gathers(0)

        def body(j, carry):
            p = lax.rem(j, nbuf)
            po = lax.rem(j, nob)

            @pl.when(j + 2 < chunks)
            def _():
                fetch_idx(j + 2)

            @pl.when(j + 1 < chunks)
            def _():
                wait_idx(j + 1)
                gathers(j + 1)

            wait_gathers(j)

            @pl.when(j >= nob)
            def _():
                wait_wo(j - nob)

            def row(rr, c2):
                for u in range(4):
                    r = rr * 4 + u
                    bufo[po, r, pl.ds(0, 16)] = (bufd[p, r, pl.ds(0, 16)]
                                                 + bufs[p, r, pl.ds(32, 16)])
                    bufo[po, r, pl.ds(16, 16)] = (bufd[p, r, pl.ds(16, 16)]
                                                  + bufs[p, r, pl.ds(48, 16)])
                    bufo[po, r, pl.ds(32, 16)] = (bufd[p, r, pl.ds(64, 16)]
                                                  - bufs[p, r, pl.ds(64, 16)])
                return c2

            lax.fori_loop(0, k // 4, row, 0)
            off_out = base - lo + j * k
            pltpu.async_copy(bufo.at[po], out_hbm.at[pl.ds(off_out, k)],
                             osem.at[po])
            return carry

        lax.fori_loop(0, chunks, body, 0)
        wait_wo(chunks - 2)
        wait_wo(chunks - 1)

    return kern(tbl, src, dst)


def _edge_body(gp_ref, ea_ref, wsm_ref, wea_ref, b0_ref, w1_ref,
               b1_ref, w2_ref, b2_ref, cw_ref, cb_ref, offs_ref, s_ref,
               ones4_ref, w0b_ref, oh4_ref, out_ref):
    gp = gp_ref[...]
    d4 = gp[:, 32:36]
    dd32 = jnp.dot(d4 * d4, ones4_ref[...],
                   preferred_element_type=jnp.float32)
    dist32 = dd32 * lax.rsqrt(dd32 + 1e-30)
    s = s_ref[0, 0]
    t = dist32 * s - offs_ref[...]
    sm = jnp.exp(-(t * t))
    pre0 = (gp[:, :32]
            + jnp.dot(sm, wsm_ref[...], preferred_element_type=jnp.float32)
            + jnp.dot(ea_ref[...], wea_ref[...], preferred_element_type=jnp.float32)
            + b0_ref[...])
    h1 = _lrelu(pre0)
    h2 = _lrelu(jnp.dot(h1, w1_ref[...], preferred_element_type=jnp.float32)
                + b1_ref[...])
    z = jnp.dot(h2, w2_ref[...], preferred_element_type=jnp.float32) + b2_ref[...]
    msg = _lrelu(z)
    m32 = jnp.dot(msg, w0b_ref[...], preferred_element_type=jnp.float32)
    ew = jnp.dot(msg, cw_ref[...], preferred_element_type=jnp.float32) \
        + cb_ref[0, 0]
    q = ew / (dist32[:, 0:1] + 1e-6)
    out_ref[:, 0:32] = m32
    out_ref[:, 32:36] = d4 * q + oh4_ref[...]


def _edge_mlp(gp, ea, wsm, wea, b0, w1, b1, w2, b2, cw, cb, offs, s,
              ones4, w0b, oh4, eb):
    e = gp.shape[0]
    de = ea.shape[1]
    g = offs.shape[1]
    dout = w2.shape[1]
    const = lambda i: (0, 0)
    return pl.pallas_call(
        _edge_body,
        grid=(e // eb,),
        in_specs=[
            pl.BlockSpec((eb, 128), lambda i: (i, 0)),
            pl.BlockSpec((eb, de), lambda i: (i, 0)),
            pl.BlockSpec((g, 32), const),
            pl.BlockSpec((de, 32), const),
            pl.BlockSpec((1, 32), const),
            pl.BlockSpec((32, 32), const),
            pl.BlockSpec((1, 32), const),
            pl.BlockSpec((32, dout), const),
            pl.BlockSpec((1, dout), const),
            pl.BlockSpec((dout, 1), const),
            pl.BlockSpec((1, 1), const),
            pl.BlockSpec((1, g), const),
            pl.BlockSpec((1, 1), const),
            pl.BlockSpec((4, g), const),
            pl.BlockSpec((dout, 32), const),
            pl.BlockSpec((1, 4), const),
        ],
        out_specs=pl.BlockSpec((eb, 128), lambda i: (i, 0)),
        out_shape=jax.ShapeDtypeStruct((e, 128), jnp.float32),
    )(gp, ea, wsm, wea, b0, w1, b1, w2, b2, cw, cb, offs, s, ones4, w0b, oh4)


def _sc_scatter(rows0_in, rows1_in, dst, n):
    half = rows0_in.shape[0]
    e = 2 * half
    k = 80
    per_w = e // (_NC * _NS)
    chunks = per_w // k
    zrows = 80
    nblk = n // zrows
    mesh = plsc.VectorSubcoreMesh(
        core_axis_name="c", subcore_axis_name="s",
        num_cores=_NC, num_subcores=_NS)

    nbuf = 3

    @functools.partial(
        pl.kernel,
        out_type=jax.ShapeDtypeStruct((_NC, n, 128), jnp.float32),
        mesh=mesh,
        scratch_types=[
            pltpu.VMEM((nbuf, k), jnp.int32),
            pltpu.VMEM((nbuf, k, 128), jnp.float32),
            pltpu.VMEM((zrows, 128), jnp.float32),
            pltpu.VMEM_SHARED((n, 128), jnp.float32),
            pltpu.SemaphoreType.DMA((nbuf,)),
            pltpu.SemaphoreType.DMA((nbuf,)),
        ],
    )
    def kern(rows0_hbm, rows1_hbm, dst_hbm, out_hbm, idx, rows, zbuf, acc,
             fsem, ssem):
        c = lax.axis_index("c")
        s = lax.axis_index("s")
        wid = s * _NC + c
        zero = jnp.zeros((16,), jnp.float32)

        def zr(r, carry):
            for cc in range(8):
                zbuf[r, pl.ds(cc * 16, 16)] = zero
            return carry

        lax.fori_loop(0, zrows, zr, 0)

        def zc(i, carry):
            blk = i * _NS + s

            @pl.when(blk < nblk)
            def _():
                pltpu.sync_copy(zbuf, acc.at[pl.ds(blk * zrows, zrows)])

            return carry

        lax.fori_loop(0, (nblk + _NS - 1) // _NS, zc, 0)
        plsc.subcore_barrier()

        base = wid * per_w

        lo_half = wid < _NS

        def fetch(j):
            p = lax.rem(j, nbuf)
            off = base + j * k
            pltpu.async_copy(dst_hbm.at[pl.ds(off, k)], idx.at[p], fsem.at[p])

            @pl.when(lo_half)
            def _():
                pltpu.async_copy(rows0_hbm.at[pl.ds(off, k)], rows.at[p],
                                 fsem.at[p])

            @pl.when(jnp.logical_not(lo_half))
            def _():
                pltpu.async_copy(rows1_hbm.at[pl.ds(off - half, k)],
                                 rows.at[p], fsem.at[p])

        def wait_fetch(j):
            p = lax.rem(j, nbuf)
            pltpu.make_async_copy(dst_hbm.at[pl.ds(0, k)], idx.at[p],
                                  fsem.at[p]).wait()
            pltpu.make_async_copy(rows0_hbm.at[pl.ds(0, k)], rows.at[p],
                                  fsem.at[p]).wait()

        def scat(j):
            p = lax.rem(j, nbuf)
            pltpu.async_copy(rows.at[p], acc.at[idx.at[p]], ssem.at[p],
                             add=True)

        def wait_scat(j):
            p = lax.rem(j, nbuf)
            pltpu.make_async_copy(rows.at[p], acc.at[idx.at[p]],
                                  ssem.at[p]).wait()

        fetch(0)
        fetch(1)

        def body(j, carry):
            @pl.when(j >= 1)
            def _():
                wait_scat(j - 1)

            @pl.when(j + 2 < chunks)
            def _():
                fetch(j + 2)

            wait_fetch(j)
            scat(j)
            return carry

        lax.fori_loop(0, chunks, body, 0)
        wait_scat(chunks - 1)
        plsc.subcore_barrier()

        def wb(i, carry):
            blk = i * _NS + s

            @pl.when(blk < nblk)
            def _():
                pltpu.sync_copy(acc.at[pl.ds(blk * zrows, zrows)],
                                out_hbm.at[c].at[pl.ds(blk * zrows, zrows)])

            return carry

        lax.fori_loop(0, (nblk + _NS - 1) // _NS, wb, 0)

    return kern(rows0_in, rows1_in, dst)


def _final_body(x_ref, pm_ref, w0a_ref, b0_ref, w1_ref,
                b1_ref, w2_ref, b2_ref, wo_ref, bo_ref, xu_ref, pu_ref):
    agg = pm_ref[0] + pm_ref[1]
    m32 = agg[:, 0:32]
    num = agg[:, 32:35]
    cnt = agg[:, 35:36]
    pu_ref[...] = num / jnp.maximum(cnt, 1.0)
    h = _lrelu(jnp.dot(x_ref[...], w0a_ref[...], preferred_element_type=jnp.float32)
               + m32 + b0_ref[...])
    h = _lrelu(jnp.dot(h, w1_ref[...], preferred_element_type=jnp.float32)
               + b1_ref[...])
    h = _lrelu(jnp.dot(h, w2_ref[...], preferred_element_type=jnp.float32)
               + b2_ref[...])
    xu_ref[...] = (jnp.dot(h, wo_ref[...], preferred_element_type=jnp.float32)
                   + bo_ref[...])


def _node_update(x, pm, w0a, b0, w1, b1, w2, b2, wo, bo, nb):
    n, d = x.shape
    hh = w0a.shape[1]
    const = lambda i: (0, 0)
    return pl.pallas_call(
        _final_body,
        grid=(n // nb,),
        in_specs=[
            pl.BlockSpec((nb, d), lambda i: (i, 0)),
            pl.BlockSpec((2, nb, 128), lambda i: (0, i, 0)),
            pl.BlockSpec((d, hh), const),
            pl.BlockSpec((1, hh), const),
            pl.BlockSpec((hh, hh), const),
            pl.BlockSpec((1, hh), const),
            pl.BlockSpec((hh, d), const),
            pl.BlockSpec((1, d), const),
            pl.BlockSpec((d, d), const),
            pl.BlockSpec((1, d), const),
        ],
        out_specs=[
            pl.BlockSpec((nb, d), lambda i: (i, 0)),
            pl.BlockSpec((nb, 3), lambda i: (i, 0)),
        ],
        out_shape=[
            jax.ShapeDtypeStruct((n, d), jnp.float32),
            jax.ShapeDtypeStruct((n, 3), jnp.float32),
        ],
    )(x, pm, w0a, b0, w1, b1, w2, b2, wo, bo)


def kernel(x, pos, edge_index, batch, edge_attr, msg_W0, msg_b0, msg_W1,
           msg_b1, msg_W2, msg_b2, coord_W, coord_b, node_W0, node_b0,
           node_W1, node_b1, node_W2, node_b2, out_W, out_b):
    n, d = x.shape
    g = msg_W0.shape[0] - 2 * d - edge_attr.shape[1]
    src = edge_index[0]
    dst = edge_index[1]

    pos4 = jnp.pad(pos, ((0, 0), (0, 1)))
    tbl = _node_table(x, pos4, msg_W0[:d], msg_W0[d:2 * d], nb=2000)

    e = edge_index.shape[1]
    half = e // 2
    gp0 = _sc_gather(tbl, src, dst, 0, half)
    gp1 = _sc_gather(tbl, src, dst, half, e)

    step = 16.0 / (g - 1)
    s = (0.5 ** 0.5) / step
    offs = (jnp.linspace(0.0, 16.0, g) * s).reshape(1, g).astype(jnp.float32)
    s_arr = jnp.full((1, 1), s, jnp.float32)

    ones4 = jnp.ones((4, g), jnp.float32)
    oh4 = jnp.array([[0.0, 0.0, 0.0, 1.0]], jnp.float32)

    def mlp_half(gp_i, ea_i):
        return _edge_mlp(
            gp_i, ea_i,
            msg_W0[2 * d:2 * d + g], msg_W0[2 * d + g:], msg_b0.reshape(1, -1),
            msg_W1, msg_b1.reshape(1, -1), msg_W2, msg_b2.reshape(1, -1),
            coord_W.reshape(d, 1), coord_b.reshape(1, 1), offs, s_arr,
            ones4, node_W0[d:], oh4, eb=8000)

    packed0 = mlp_half(gp0, edge_attr[:half])
    packed1 = mlp_half(gp1, edge_attr[half:])

    pm = _sc_scatter(packed0, packed1, dst, n)

    x_update, pos_update = _node_update(
        x, pm, node_W0[:d], node_b0.reshape(1, -1),
        node_W1, node_b1.reshape(1, -1), node_W2, node_b2.reshape(1, -1),
        out_W, out_b.reshape(1, -1), nb=2000)
    return (x_update, pos_update)
